# baseline (speedup 1.0000x reference)
import functools
import jax
import jax.numpy as jnp
from jax import lax
from jax.experimental import pallas as pl
from jax.experimental.pallas import tpu as pltpu
from jax.experimental.pallas import tpu_sc as plsc

B, T, C = 1, 2048, 768
E, K, G, TG = 64, 8, 8, 4
H, SH = 384, 1536
S = B * T
SK = S * K
EPG = E // G
R = 128
NB = E + SK // R
NPAD = NB * R
NEG = -1e30

NC, NS, L = 2, 16, 16
NW = NC * NS


def _router_body(x_ref, rw_ref, eb_ref, pos_ref, fw_ref, be_ref, nba_ref):
    x = x_ref[...]
    rw = rw_ref[...]
    logits = lax.dot_general(x, rw, (((1,), (1,)), ((), ())),
                             preferred_element_type=jnp.float32)
    scores = jax.nn.sigmoid(logits)
    sb = scores + eb_ref[...]

    sb3 = sb.reshape(S, G, EPG)
    io3 = lax.broadcasted_iota(jnp.int32, (S, G, EPG), 2)
    m1 = jnp.max(sb3, axis=2)
    a1 = jnp.min(jnp.where(sb3 == m1[:, :, None], io3, EPG), axis=2)
    m2 = jnp.max(jnp.where(io3 == a1[:, :, None], NEG, sb3), axis=2)
    group_scores = m1 + m2

    iog = lax.broadcasted_iota(jnp.int32, (S, G), 1)
    gs = group_scores
    gmask = jnp.zeros((S, G), jnp.float32)
    for _ in range(TG):
        m = jnp.max(gs, axis=1)
        a = jnp.min(jnp.where(gs == m[:, None], iog, G), axis=1)
        sel = iog == a[:, None]
        gmask = jnp.where(sel, 1.0, gmask)
        gs = jnp.where(sel, NEG, gs)
    score_mask = jnp.repeat(gmask, EPG, axis=1)
    masked = jnp.where(score_mask == 0, NEG, sb)

    ioe = lax.broadcasted_iota(jnp.int32, (S, E), 1)
    topk = []
    fws = []
    for _ in range(K):
        m = jnp.max(masked, axis=1)
        a = jnp.min(jnp.where(masked == m[:, None], ioe, E), axis=1)
        sel = ioe == a[:, None]
        topk.append(a)
        fws.append(jnp.sum(jnp.where(sel, scores, 0.0), axis=1))
        masked = jnp.where(sel, NEG, masked)
    fw = jnp.stack(fws, axis=1)
    fw = fw / (jnp.sum(fw, axis=1, keepdims=True) + 1e-20)
    fw_ref[...] = fw

    hist = jnp.zeros((S, E), jnp.float32)
    for k in range(K):
        hist = hist + jnp.where(ioe == topk[k][:, None], 1.0, 0.0)
    ior = lax.broadcasted_iota(jnp.int32, (S, S), 0)
    ioc = lax.broadcasted_iota(jnp.int32, (S, S), 1)
    tri = jnp.where(ior > ioc, 1.0, 0.0)
    csum_excl = lax.dot_general(tri, hist, (((1,), (0,)), ((), ())),
                                preferred_element_type=jnp.float32)
    counts = jnp.sum(hist, axis=0, keepdims=True).astype(jnp.int32)
    pc = ((counts + (R - 1)) >> 7) << 7
    ie1 = lax.broadcasted_iota(jnp.int32, (E, E), 0)
    ie2 = lax.broadcasted_iota(jnp.int32, (E, E), 1)
    trie = jnp.where(ie1 < ie2, 1.0, 0.0)
    start = lax.dot_general(pc.astype(jnp.float32), trie,
                            (((1,), (0,)), ((), ())),
                            preferred_element_type=jnp.float32)

    slot_base = start + csum_excl
    pos_cols = []
    for k in range(K):
        sel = ioe == topk[k][:, None]
        pos_k = jnp.sum(jnp.where(sel, slot_base, 0.0), axis=1)
        pos_cols.append(pos_k.astype(jnp.int32))
    pos_ref[...] = jnp.stack(pos_cols, axis=1)

    start_i = start.astype(jnp.int32)
    iob = lax.broadcasted_iota(jnp.int32, (E, NB), 1)
    cmp = jnp.where(start_i.reshape(E, 1) <= iob * R, 1, 0)
    be = jnp.sum(cmp, axis=0, keepdims=True) - 1
    be_ref[...] = jnp.clip(be, 0, E - 1).astype(jnp.int32)
    total = jnp.sum(pc, axis=1, keepdims=True)
    nba_ref[...] = total >> 7


def _run_router(x_flat, router_w, e_bias):
    return pl.pallas_call(
        _router_body,
        out_shape=[
            jax.ShapeDtypeStruct((S, K), jnp.int32),
            jax.ShapeDtypeStruct((S, K), jnp.float32),
            jax.ShapeDtypeStruct((1, NB), jnp.int32),
            jax.ShapeDtypeStruct((1, 1), jnp.int32),
        ],
    )(x_flat, router_w, e_bias.reshape(1, E))


def _builder_body(pos_hbm, fw_hbm, src_hbm, wgt_hbm,
                  pos_v, fw_v, src_v, wgt_v):
    cid = lax.axis_index("c")
    sid = lax.axis_index("s")

    @pl.when(jnp.logical_and(cid == 0, sid == 0))
    def _():
        pltpu.sync_copy(pos_hbm, pos_v)
        pltpu.sync_copy(fw_hbm, fw_v)
        zi = jnp.zeros((L,), jnp.int32)
        zf = jnp.zeros((L,), jnp.float32)

        def zero(i, carry):
            src_v[pl.ds(i * L, L)] = zi
            wgt_v[pl.ds(i * L, L)] = zf
            return carry

        lax.fori_loop(0, NPAD // L, zero, 0)

        lane = lax.iota(jnp.int32, L)

        def scat(i, carry):
            base = i * L
            idx = pos_v[pl.ds(base, L)]
            vals = fw_v[pl.ds(base, L)]
            tok = lax.shift_right_logical(base + lane, 3)
            plsc.store_scatter(src_v, [idx], tok)
            plsc.store_scatter(wgt_v, [idx], vals)
            return carry

        lax.fori_loop(0, SK // L, scat, 0)
        pltpu.sync_copy(src_v, src_hbm)
        pltpu.sync_copy(wgt_v, wgt_hbm)


def _run_builder(pos_flat, fw_flat):
    mesh = plsc.VectorSubcoreMesh(core_axis_name="c", subcore_axis_name="s", num_cores=NC, num_subcores=NS)
    return pl.kernel(
        _builder_body,
        compiler_params=pltpu.CompilerParams(needs_layout_passes=False),
        out_type=[
            jax.ShapeDtypeStruct((NPAD,), jnp.int32),
            jax.ShapeDtypeStruct((NPAD,), jnp.float32),
        ],
        mesh=mesh,
        scratch_types=[
            pltpu.VMEM((SK,), jnp.int32),
            pltpu.VMEM((SK,), jnp.float32),
            pltpu.VMEM((NPAD,), jnp.int32),
            pltpu.VMEM((NPAD,), jnp.float32),
        ],
    )(pos_flat, fw_flat)


_G_CH = 64
_G_PER_W = NPAD // NW


_G_NCH = _G_PER_W // _G_CH


def _gather_body(x_hbm, src_hbm, xs_hbm, idx_v, b0, b1, g0, g1, o0, o1):
    wid = lax.axis_index("s") * NC + lax.axis_index("c")
    base = wid * _G_PER_W
    bufs, gsems, osems = (b0, b1), (g0, g1), (o0, o1)

    pltpu.sync_copy(src_hbm.at[pl.ds(base, _G_PER_W)], idx_v)

    def start_gather(j):
        return pltpu.async_copy(x_hbm.at[idx_v.at[pl.ds(j * _G_CH, _G_CH)]],
                                bufs[j % 2], gsems[j % 2])

    def start_out(j):
        return pltpu.async_copy(bufs[j % 2],
                                xs_hbm.at[pl.ds(base + j * _G_CH, _G_CH)],
                                osems[j % 2])

    gd = [None] * _G_NCH
    od = [None] * _G_NCH
    gd[0] = start_gather(0)
    for j in range(_G_NCH):
        if j + 1 < _G_NCH:
            if j >= 1:
                od[j - 1].wait()
            gd[j + 1] = start_gather(j + 1)
        gd[j].wait()
        od[j] = start_out(j)
    od[_G_NCH - 2].wait()
    od[_G_NCH - 1].wait()


def _run_gather(x_flat, src):
    mesh = plsc.VectorSubcoreMesh(core_axis_name="c", subcore_axis_name="s", num_cores=NC, num_subcores=NS)
    return pl.kernel(
        _gather_body,
        out_type=jax.ShapeDtypeStruct((NPAD, C), jnp.float32),
        mesh=mesh,
        scratch_types=[
            pltpu.VMEM((_G_PER_W,), jnp.int32),
            pltpu.VMEM((_G_CH, C), jnp.float32),
            pltpu.VMEM((_G_CH, C), jnp.float32),
            pltpu.SemaphoreType.DMA,
            pltpu.SemaphoreType.DMA,
            pltpu.SemaphoreType.DMA,
            pltpu.SemaphoreType.DMA,
        ],
    )(x_flat, src)


def _expert_body(be_ref, nba_ref, xs_ref, g_ref, u_ref, d_ref, w_ref, ys_ref):
    b = pl.program_id(0)

    @pl.when(b < nba_ref[0])
    def _():
        xb = xs_ref[...]
        g = jnp.dot(xb, g_ref[0], preferred_element_type=jnp.float32)
        u = jnp.dot(xb, u_ref[0], preferred_element_type=jnp.float32)
        h = g * jax.nn.sigmoid(g) * u
        y = jnp.dot(h, d_ref[0], preferred_element_type=jnp.float32)
        w = w_ref[0, 0, :]
        ys_ref[...] = y * w[:, None]


def _run_experts(xs, wgt_r, gate_weight, up_weight, down_weight,
                 block_expert, nb_active):
    def clamp(b, nba):
        return jnp.minimum(b, nba[0] - 1)

    grid_spec = pltpu.PrefetchScalarGridSpec(
        num_scalar_prefetch=2,
        grid=(NB,),
        in_specs=[
            pl.BlockSpec((R, C), lambda b, be, nba: (clamp(b, nba), 0)),
            pl.BlockSpec((1, C, H), lambda b, be, nba: (be[clamp(b, nba)], 0, 0)),
            pl.BlockSpec((1, C, H), lambda b, be, nba: (be[clamp(b, nba)], 0, 0)),
            pl.BlockSpec((1, H, C), lambda b, be, nba: (be[clamp(b, nba)], 0, 0)),
            pl.BlockSpec((1, 1, R), lambda b, be, nba: (clamp(b, nba), 0, 0)),
        ],
        out_specs=pl.BlockSpec((R, C), lambda b, be, nba: (clamp(b, nba), 0)),
    )
    return pl.pallas_call(
        _expert_body,
        grid_spec=grid_spec,
        out_shape=jax.ShapeDtypeStruct((NPAD, C), jnp.float32),
    )(block_expert, nb_active, xs, gate_weight, up_weight, down_weight, wgt_r)


_SH_BT = 256


def _shared_body(x_ref, gw_ref, uw_ref, dw_ref, o_ref):
    xb = x_ref[...]
    g = lax.dot_general(xb, gw_ref[...], (((1,), (1,)), ((), ())),
                        preferred_element_type=jnp.float32)
    u = lax.dot_general(xb, uw_ref[...], (((1,), (1,)), ((), ())),
                        preferred_element_type=jnp.float32)
    h = g * jax.nn.sigmoid(g) * u
    o_ref[...] = lax.dot_general(h, dw_ref[...], (((1,), (1,)), ((), ())),
                                 preferred_element_type=jnp.float32)


def _run_shared(x_flat, shared_gate_w, shared_up_w, shared_down_w):
    return pl.pallas_call(
        _shared_body,
        grid=(S // _SH_BT,),
        in_specs=[
            pl.BlockSpec((_SH_BT, C), lambda t: (t, 0)),
            pl.BlockSpec((SH, C), lambda t: (0, 0)),
            pl.BlockSpec((SH, C), lambda t: (0, 0)),
            pl.BlockSpec((C, SH), lambda t: (0, 0)),
        ],
        out_specs=pl.BlockSpec((_SH_BT, C), lambda t: (t, 0)),
        out_shape=jax.ShapeDtypeStruct((S, C), jnp.float32),
    )(x_flat, shared_gate_w, shared_up_w, shared_down_w)


_C_TPW = S // NW
_C_TCH = 8


def _combine_body(ys_hbm, pos_hbm, sh_hbm, out_hbm,
                  idx_v, rows_v, sh_v, out_v, sem):
    wid = lax.axis_index("s") * NC + lax.axis_index("c")

    def chunk(cjk, carry):
        tok0 = wid * _C_TPW + cjk * _C_TCH
        pltpu.sync_copy(pos_hbm.at[pl.ds(tok0 * K, _C_TCH * K)], idx_v)
        cp = pltpu.async_copy(ys_hbm.at[idx_v], rows_v, sem)
        pltpu.sync_copy(sh_hbm.at[pl.ds(tok0, _C_TCH)], sh_v)
        cp.wait()
        for t in range(_C_TCH):
            def lanes(v, carry2):
                sl = pl.ds(v * L, L)
                acc = sh_v[t, sl]
                for j in range(K):
                    acc = acc + rows_v[t * K + j, sl]
                out_v[t, sl] = acc
                return carry2

            lax.fori_loop(0, C // L, lanes, 0)
        pltpu.sync_copy(out_v, out_hbm.at[pl.ds(tok0, _C_TCH)])
        return carry

    lax.fori_loop(0, _C_TPW // _C_TCH, chunk, 0)


def _run_combine(ys, pos_flat, shared_out):
    mesh = plsc.VectorSubcoreMesh(core_axis_name="c", subcore_axis_name="s", num_cores=NC, num_subcores=NS)
    return pl.kernel(
        _combine_body,
        out_type=jax.ShapeDtypeStruct((S, C), jnp.float32),
        mesh=mesh,
        scratch_types=[
            pltpu.VMEM((_C_TCH * K,), jnp.int32),
            pltpu.VMEM((_C_TCH * K, C), jnp.float32),
            pltpu.VMEM((_C_TCH, C), jnp.float32),
            pltpu.VMEM((_C_TCH, C), jnp.float32),
            pltpu.SemaphoreType.DMA,
        ],
    )(ys, pos_flat, shared_out)


@jax.jit
def kernel(x, router_w, e_bias, gate_weight, up_weight, down_weight,
           shared_gate_w, shared_up_w, shared_down_w):
    x_flat = x.reshape(S, C)
    pos, fw, block_expert, nb_active = _run_router(x_flat, router_w, e_bias)
    pos_flat = pos.reshape(SK)
    src, wgt = _run_builder(pos_flat, fw.reshape(SK))
    xs = _run_gather(x_flat, src)
    shared_out = _run_shared(x_flat, shared_gate_w, shared_up_w, shared_down_w)
    ys = _run_experts(xs, wgt.reshape(NB, 1, R), gate_weight, up_weight,
                      down_weight, block_expert.reshape(NB), nb_active.reshape(1))
    out = _run_combine(ys, pos_flat, shared_out)
    return out.reshape(B, T, C)


# speedup vs baseline: 2.1252x; 2.1252x over previous
import functools
import jax
import jax.numpy as jnp
from jax import lax
from jax.experimental import pallas as pl
from jax.experimental.pallas import tpu as pltpu
from jax.experimental.pallas import tpu_sc as plsc

B, T, C = 1, 2048, 768
E, K, G, TG = 64, 8, 8, 4
H, SH = 384, 1536
S = B * T
SK = S * K
EPG = E // G
R = 128
NB = E + SK // R
NPAD = NB * R
NEG = -1e30

NC, NS, L = 2, 16, 16
NW = NC * NS


def _router_body(x_ref, rw_ref, eb_ref, pos_ref, fw_ref, be_ref, nba_ref):
    x = x_ref[...]
    rw = rw_ref[...]
    logits = lax.dot_general(x, rw, (((1,), (1,)), ((), ())),
                             preferred_element_type=jnp.float32)
    scores = jax.nn.sigmoid(logits)
    sb = scores + eb_ref[...]

    sb3 = sb.reshape(S, G, EPG)
    io3 = lax.broadcasted_iota(jnp.int32, (S, G, EPG), 2)
    m1 = jnp.max(sb3, axis=2)
    a1 = jnp.min(jnp.where(sb3 == m1[:, :, None], io3, EPG), axis=2)
    m2 = jnp.max(jnp.where(io3 == a1[:, :, None], NEG, sb3), axis=2)
    group_scores = m1 + m2

    iog = lax.broadcasted_iota(jnp.int32, (S, G), 1)
    gs = group_scores
    gmask = jnp.zeros((S, G), jnp.float32)
    for _ in range(TG):
        m = jnp.max(gs, axis=1)
        a = jnp.min(jnp.where(gs == m[:, None], iog, G), axis=1)
        sel = iog == a[:, None]
        gmask = jnp.where(sel, 1.0, gmask)
        gs = jnp.where(sel, NEG, gs)
    score_mask = jnp.repeat(gmask, EPG, axis=1)
    masked = jnp.where(score_mask == 0, NEG, sb)

    ioe = lax.broadcasted_iota(jnp.int32, (S, E), 1)
    topk = []
    fws = []
    for _ in range(K):
        m = jnp.max(masked, axis=1)
        a = jnp.min(jnp.where(masked == m[:, None], ioe, E), axis=1)
        sel = ioe == a[:, None]
        topk.append(a)
        fws.append(jnp.sum(jnp.where(sel, scores, 0.0), axis=1))
        masked = jnp.where(sel, NEG, masked)
    fw = jnp.stack(fws, axis=1)
    fw = fw / (jnp.sum(fw, axis=1, keepdims=True) + 1e-20)
    fw_ref[...] = fw

    hist = jnp.zeros((S, E), jnp.float32)
    for k in range(K):
        hist = hist + jnp.where(ioe == topk[k][:, None], 1.0, 0.0)
    ior = lax.broadcasted_iota(jnp.int32, (S, S), 0)
    ioc = lax.broadcasted_iota(jnp.int32, (S, S), 1)
    tri = jnp.where(ior > ioc, 1.0, 0.0)
    csum_excl = lax.dot_general(tri, hist, (((1,), (0,)), ((), ())),
                                preferred_element_type=jnp.float32)
    counts = jnp.sum(hist, axis=0, keepdims=True).astype(jnp.int32)
    pc = ((counts + (R - 1)) >> 7) << 7
    ie1 = lax.broadcasted_iota(jnp.int32, (E, E), 0)
    ie2 = lax.broadcasted_iota(jnp.int32, (E, E), 1)
    trie = jnp.where(ie1 < ie2, 1.0, 0.0)
    start = lax.dot_general(pc.astype(jnp.float32), trie,
                            (((1,), (0,)), ((), ())),
                            preferred_element_type=jnp.float32)

    slot_base = start + csum_excl
    pos_cols = []
    for k in range(K):
        sel = ioe == topk[k][:, None]
        pos_k = jnp.sum(jnp.where(sel, slot_base, 0.0), axis=1)
        pos_cols.append(pos_k.astype(jnp.int32))
    pos_ref[...] = jnp.stack(pos_cols, axis=1)

    start_i = start.astype(jnp.int32)
    iob = lax.broadcasted_iota(jnp.int32, (E, NB), 1)
    cmp = jnp.where(start_i.reshape(E, 1) <= iob * R, 1, 0)
    be = jnp.sum(cmp, axis=0, keepdims=True) - 1
    be_ref[...] = jnp.clip(be, 0, E - 1).astype(jnp.int32)
    total = jnp.sum(pc, axis=1, keepdims=True)
    nba_ref[...] = total >> 7


def _run_router(x_flat, router_w, e_bias):
    return pl.pallas_call(
        _router_body,
        out_shape=[
            jax.ShapeDtypeStruct((S, K), jnp.int32),
            jax.ShapeDtypeStruct((S, K), jnp.float32),
            jax.ShapeDtypeStruct((1, NB), jnp.int32),
            jax.ShapeDtypeStruct((1, 1), jnp.int32),
        ],
    )(x_flat, router_w, e_bias.reshape(1, E))


def _builder_body(pos_hbm, fw_hbm, wgt_hbm, pos_v, fw_v, wgt_v):
    cid = lax.axis_index("c")
    sid = lax.axis_index("s")

    @pl.when(jnp.logical_and(cid == 0, sid == 0))
    def _():
        pltpu.sync_copy(pos_hbm, pos_v)
        pltpu.sync_copy(fw_hbm, fw_v)
        zf = jnp.zeros((L,), jnp.float32)

        def zero(i, carry):
            wgt_v[pl.ds(i * L, L)] = zf
            return carry

        lax.fori_loop(0, NPAD // L, zero, 0)

        def scat(i, carry):
            base = i * L
            idx = pos_v[pl.ds(base, L)]
            vals = fw_v[pl.ds(base, L)]
            plsc.store_scatter(wgt_v, [idx], vals)
            return carry

        lax.fori_loop(0, SK // L, scat, 0)
        pltpu.sync_copy(wgt_v, wgt_hbm)


def _run_builder(pos_flat, fw_flat):
    mesh = plsc.VectorSubcoreMesh(core_axis_name="c", subcore_axis_name="s", num_cores=NC, num_subcores=NS)
    return pl.kernel(
        _builder_body,
        compiler_params=pltpu.CompilerParams(needs_layout_passes=False),
        out_type=jax.ShapeDtypeStruct((NPAD,), jnp.float32),
        mesh=mesh,
        scratch_types=[
            pltpu.VMEM((SK,), jnp.int32),
            pltpu.VMEM((SK,), jnp.float32),
            pltpu.VMEM((NPAD,), jnp.float32),
        ],
    )(pos_flat, fw_flat)


_D_TPW = S // NW


def _dispatch_body(x_hbm, pos3_hbm, xs_hbm, idx_v, buf, sem):
    wid = lax.axis_index("s") * NC + lax.axis_index("c")
    t0 = wid * _D_TPW
    pltpu.sync_copy(pos3_hbm.at[wid], idx_v)
    pltpu.sync_copy(x_hbm.at[pl.ds(t0, _D_TPW)], buf)
    cps = [pltpu.async_copy(buf, xs_hbm.at[idx_v.at[k]], sem)
           for k in range(K)]
    for cp in cps:
        cp.wait()


def _run_dispatch(x_flat, pos3):
    mesh = plsc.VectorSubcoreMesh(core_axis_name="c", subcore_axis_name="s", num_cores=NC, num_subcores=NS)
    return pl.kernel(
        _dispatch_body,
        out_type=jax.ShapeDtypeStruct((NPAD, C), jnp.float32),
        mesh=mesh,
        scratch_types=[
            pltpu.VMEM((K, _D_TPW), jnp.int32),
            pltpu.VMEM((_D_TPW, C), jnp.float32),
            pltpu.SemaphoreType.DMA,
        ],
    )(x_flat, pos3)


def _expert_body(be_ref, nba_ref, xs_ref, g_ref, u_ref, d_ref, w_ref, ys_ref):
    b = pl.program_id(0)

    @pl.when(b < nba_ref[0])
    def _():
        xb = xs_ref[...]
        g = jnp.dot(xb, g_ref[0], preferred_element_type=jnp.float32)
        u = jnp.dot(xb, u_ref[0], preferred_element_type=jnp.float32)
        h = g * jax.nn.sigmoid(g) * u
        y = jnp.dot(h, d_ref[0], preferred_element_type=jnp.float32)
        w = w_ref[0, 0, :]
        ys_ref[...] = y * w[:, None]


def _run_experts(xs, wgt_r, gate_weight, up_weight, down_weight,
                 block_expert, nb_active):
    def clamp(b, nba):
        return jnp.minimum(b, nba[0] - 1)

    grid_spec = pltpu.PrefetchScalarGridSpec(
        num_scalar_prefetch=2,
        grid=(NB,),
        in_specs=[
            pl.BlockSpec((R, C), lambda b, be, nba: (clamp(b, nba), 0)),
            pl.BlockSpec((1, C, H), lambda b, be, nba: (be[clamp(b, nba)], 0, 0)),
            pl.BlockSpec((1, C, H), lambda b, be, nba: (be[clamp(b, nba)], 0, 0)),
            pl.BlockSpec((1, H, C), lambda b, be, nba: (be[clamp(b, nba)], 0, 0)),
            pl.BlockSpec((1, 1, R), lambda b, be, nba: (clamp(b, nba), 0, 0)),
        ],
        out_specs=pl.BlockSpec((R, C), lambda b, be, nba: (clamp(b, nba), 0)),
    )
    return pl.pallas_call(
        _expert_body,
        grid_spec=grid_spec,
        out_shape=jax.ShapeDtypeStruct((NPAD, C), jnp.float32),
    )(block_expert, nb_active, xs, gate_weight, up_weight, down_weight, wgt_r)


_SH_BT = 256


def _shared_body(x_ref, gw_ref, uw_ref, dw_ref, o_ref):
    xb = x_ref[...]
    g = lax.dot_general(xb, gw_ref[...], (((1,), (1,)), ((), ())),
                        preferred_element_type=jnp.float32)
    u = lax.dot_general(xb, uw_ref[...], (((1,), (1,)), ((), ())),
                        preferred_element_type=jnp.float32)
    h = g * jax.nn.sigmoid(g) * u
    o_ref[...] = lax.dot_general(h, dw_ref[...], (((1,), (1,)), ((), ())),
                                 preferred_element_type=jnp.float32)


def _run_shared(x_flat, shared_gate_w, shared_up_w, shared_down_w):
    return pl.pallas_call(
        _shared_body,
        grid=(S // _SH_BT,),
        in_specs=[
            pl.BlockSpec((_SH_BT, C), lambda t: (t, 0)),
            pl.BlockSpec((SH, C), lambda t: (0, 0)),
            pl.BlockSpec((SH, C), lambda t: (0, 0)),
            pl.BlockSpec((C, SH), lambda t: (0, 0)),
        ],
        out_specs=pl.BlockSpec((_SH_BT, C), lambda t: (t, 0)),
        out_shape=jax.ShapeDtypeStruct((S, C), jnp.float32),
    )(x_flat, shared_gate_w, shared_up_w, shared_down_w)


_C_TPW = S // NW
_C_TCH = 8


def _combine_body(ys_hbm, pos_hbm, sh_hbm, out_hbm,
                  idx_v, rows_v, sh_v, out_v, sem):
    wid = lax.axis_index("s") * NC + lax.axis_index("c")

    def chunk(cjk, carry):
        tok0 = wid * _C_TPW + cjk * _C_TCH
        pltpu.sync_copy(pos_hbm.at[pl.ds(tok0 * K, _C_TCH * K)], idx_v)
        cp = pltpu.async_copy(ys_hbm.at[idx_v], rows_v, sem)
        pltpu.sync_copy(sh_hbm.at[pl.ds(tok0, _C_TCH)], sh_v)
        cp.wait()
        for t in range(_C_TCH):
            def lanes(v, carry2):
                sl = pl.ds(v * L, L)
                acc = sh_v[t, sl]
                for j in range(K):
                    acc = acc + rows_v[t * K + j, sl]
                out_v[t, sl] = acc
                return carry2

            lax.fori_loop(0, C // L, lanes, 0)
        pltpu.sync_copy(out_v, out_hbm.at[pl.ds(tok0, _C_TCH)])
        return carry

    lax.fori_loop(0, _C_TPW // _C_TCH, chunk, 0)


def _run_combine(ys, pos_flat, shared_out):
    mesh = plsc.VectorSubcoreMesh(core_axis_name="c", subcore_axis_name="s", num_cores=NC, num_subcores=NS)
    return pl.kernel(
        _combine_body,
        out_type=jax.ShapeDtypeStruct((S, C), jnp.float32),
        mesh=mesh,
        scratch_types=[
            pltpu.VMEM((_C_TCH * K,), jnp.int32),
            pltpu.VMEM((_C_TCH * K, C), jnp.float32),
            pltpu.VMEM((_C_TCH, C), jnp.float32),
            pltpu.VMEM((_C_TCH, C), jnp.float32),
            pltpu.SemaphoreType.DMA,
        ],
    )(ys, pos_flat, shared_out)


@jax.jit
def kernel(x, router_w, e_bias, gate_weight, up_weight, down_weight,
           shared_gate_w, shared_up_w, shared_down_w):
    x_flat = x.reshape(S, C)
    pos, fw, block_expert, nb_active = _run_router(x_flat, router_w, e_bias)
    pos_flat = pos.reshape(SK)
    wgt = _run_builder(pos_flat, fw.reshape(SK))
    pos3 = pos.T.reshape(K, NW, _D_TPW).transpose(1, 0, 2)
    xs = _run_dispatch(x_flat, pos3)
    shared_out = _run_shared(x_flat, shared_gate_w, shared_up_w, shared_down_w)
    ys = _run_experts(xs, wgt.reshape(NB, 1, R), gate_weight, up_weight,
                      down_weight, block_expert.reshape(NB), nb_active.reshape(1))
    out = _run_combine(ys, pos_flat, shared_out)
    return out.reshape(B, T, C)


# speedup vs baseline: 2.2156x; 1.0425x over previous
import functools
import jax
import jax.numpy as jnp
from jax import lax
from jax.experimental import pallas as pl
from jax.experimental.pallas import tpu as pltpu
from jax.experimental.pallas import tpu_sc as plsc

B, T, C = 1, 2048, 768
E, K, G, TG = 64, 8, 8, 4
H, SH = 384, 1536
S = B * T
SK = S * K
EPG = E // G
R = 128
NB = E + SK // R
NPAD = NB * R
NEG = -1e30

NC, NS, L = 2, 16, 16
NW = NC * NS


def _router_body(x_ref, rw_ref, eb_ref, pos_ref, fwb_ref, xpk_ref, be_ref, nba_ref):
    x = x_ref[...]
    rw = rw_ref[...]
    logits = lax.dot_general(x, rw, (((1,), (1,)), ((), ())),
                             preferred_element_type=jnp.float32)
    scores = jax.nn.sigmoid(logits)
    sb = scores + eb_ref[...]

    sb3 = sb.reshape(S, G, EPG)
    io3 = lax.broadcasted_iota(jnp.int32, (S, G, EPG), 2)
    m1 = jnp.max(sb3, axis=2)
    a1 = jnp.min(jnp.where(sb3 == m1[:, :, None], io3, EPG), axis=2)
    m2 = jnp.max(jnp.where(io3 == a1[:, :, None], NEG, sb3), axis=2)
    group_scores = m1 + m2

    iog = lax.broadcasted_iota(jnp.int32, (S, G), 1)
    gs = group_scores
    gmask = jnp.zeros((S, G), jnp.float32)
    for _ in range(TG):
        m = jnp.max(gs, axis=1)
        a = jnp.min(jnp.where(gs == m[:, None], iog, G), axis=1)
        sel = iog == a[:, None]
        gmask = jnp.where(sel, 1.0, gmask)
        gs = jnp.where(sel, NEG, gs)
    score_mask = jnp.repeat(gmask, EPG, axis=1)
    masked = jnp.where(score_mask == 0, NEG, sb)

    ioe = lax.broadcasted_iota(jnp.int32, (S, E), 1)
    topk = []
    fws = []
    for _ in range(K):
        m = jnp.max(masked, axis=1)
        a = jnp.min(jnp.where(masked == m[:, None], ioe, E), axis=1)
        sel = ioe == a[:, None]
        topk.append(a)
        fws.append(jnp.sum(jnp.where(sel, scores, 0.0), axis=1))
        masked = jnp.where(sel, NEG, masked)
    fw = jnp.stack(fws, axis=1)
    fw = fw / (jnp.sum(fw, axis=1, keepdims=True) + 1e-20)
    fwb_ref[...] = jnp.repeat(fw, 16, axis=1)
    xpk_ref[...] = pltpu.pack_elementwise(
        [x[:, :C // 2], x[:, C // 2:]], packed_dtype=jnp.bfloat16)

    hist = jnp.zeros((S, E), jnp.float32)
    for k in range(K):
        hist = hist + jnp.where(ioe == topk[k][:, None], 1.0, 0.0)
    ior = lax.broadcasted_iota(jnp.int32, (S, S), 0)
    ioc = lax.broadcasted_iota(jnp.int32, (S, S), 1)
    tri = jnp.where(ior > ioc, 1.0, 0.0)
    csum_excl = lax.dot_general(tri, hist, (((1,), (0,)), ((), ())),
                                preferred_element_type=jnp.float32)
    counts = jnp.sum(hist, axis=0, keepdims=True).astype(jnp.int32)
    pc = ((counts + (R - 1)) >> 7) << 7
    ie1 = lax.broadcasted_iota(jnp.int32, (E, E), 0)
    ie2 = lax.broadcasted_iota(jnp.int32, (E, E), 1)
    trie = jnp.where(ie1 < ie2, 1.0, 0.0)
    start = lax.dot_general(pc.astype(jnp.float32), trie,
                            (((1,), (0,)), ((), ())),
                            preferred_element_type=jnp.float32)

    slot_base = start + csum_excl
    pos_cols = []
    for k in range(K):
        sel = ioe == topk[k][:, None]
        pos_k = jnp.sum(jnp.where(sel, slot_base, 0.0), axis=1)
        pos_cols.append(pos_k.astype(jnp.int32))
    pos_ref[...] = jnp.stack(pos_cols, axis=1)

    start_i = start.astype(jnp.int32)
    iob = lax.broadcasted_iota(jnp.int32, (E, NB), 1)
    cmp = jnp.where(start_i.reshape(E, 1) <= iob * R, 1, 0)
    be = jnp.sum(cmp, axis=0, keepdims=True) - 1
    be_ref[...] = jnp.clip(be, 0, E - 1).astype(jnp.int32)
    total = jnp.sum(pc, axis=1, keepdims=True)
    nba_ref[...] = total >> 7


def _run_router(x_flat, router_w, e_bias):
    return pl.pallas_call(
        _router_body,
        out_shape=[
            jax.ShapeDtypeStruct((S, K), jnp.int32),
            jax.ShapeDtypeStruct((S, K * 16), jnp.float32),
            jax.ShapeDtypeStruct((S, C // 2), jnp.int32),
            jax.ShapeDtypeStruct((1, NB), jnp.int32),
            jax.ShapeDtypeStruct((1, 1), jnp.int32),
        ],
    )(x_flat, router_w, e_bias.reshape(1, E))


_D_TPW = S // NW


def _dispatch_body(x_hbm, pos3_hbm, xs_hbm, idx_v, buf, sem):
    wid = lax.axis_index("s") * NC + lax.axis_index("c")
    t0 = wid * _D_TPW
    pltpu.sync_copy(pos3_hbm.at[wid], idx_v)
    pltpu.sync_copy(x_hbm.at[pl.ds(t0, _D_TPW)], buf)
    cps = [pltpu.async_copy(buf, xs_hbm.at[idx_v.at[k]], sem)
           for k in range(K)]
    for cp in cps:
        cp.wait()


def _run_dispatch(x_bf16, pos3):
    mesh = plsc.VectorSubcoreMesh(core_axis_name="c", subcore_axis_name="s", num_cores=NC, num_subcores=NS)
    return pl.kernel(
        _dispatch_body,
        out_type=jax.ShapeDtypeStruct((NPAD, C // 2), jnp.int32),
        mesh=mesh,
        scratch_types=[
            pltpu.VMEM((K, _D_TPW), jnp.int32),
            pltpu.VMEM((_D_TPW, C // 2), jnp.int32),
            pltpu.SemaphoreType.DMA,
        ],
    )(x_bf16, pos3)


def _expert_body(be_ref, nba_ref, xs_ref, g_ref, u_ref, d_ref, ys_ref):
    b = pl.program_id(0)

    @pl.when(b < nba_ref[0])
    def _():
        xpk = xs_ref[...]
        xa = pltpu.unpack_elementwise(
            xpk, index=0, packed_dtype=jnp.bfloat16, unpacked_dtype=jnp.float32)
        xb_hi = pltpu.unpack_elementwise(
            xpk, index=1, packed_dtype=jnp.bfloat16, unpacked_dtype=jnp.float32)
        xb = jnp.concatenate([xa, xb_hi], axis=1)
        g = jnp.dot(xb, g_ref[0], preferred_element_type=jnp.float32)
        u = jnp.dot(xb, u_ref[0], preferred_element_type=jnp.float32)
        h = g * jax.nn.sigmoid(g) * u
        ys_ref[...] = jnp.dot(h, d_ref[0], preferred_element_type=jnp.float32)


def _run_experts(xs, gate_weight, up_weight, down_weight,
                 block_expert, nb_active):
    def clamp(b, nba):
        return jnp.minimum(b, nba[0] - 1)

    grid_spec = pltpu.PrefetchScalarGridSpec(
        num_scalar_prefetch=2,
        grid=(NB,),
        in_specs=[
            pl.BlockSpec((R, C // 2), lambda b, be, nba: (clamp(b, nba), 0)),
            pl.BlockSpec((1, C, H), lambda b, be, nba: (be[clamp(b, nba)], 0, 0)),
            pl.BlockSpec((1, C, H), lambda b, be, nba: (be[clamp(b, nba)], 0, 0)),
            pl.BlockSpec((1, H, C), lambda b, be, nba: (be[clamp(b, nba)], 0, 0)),
        ],
        out_specs=pl.BlockSpec((R, C), lambda b, be, nba: (clamp(b, nba), 0)),
    )
    return pl.pallas_call(
        _expert_body,
        grid_spec=grid_spec,
        out_shape=jax.ShapeDtypeStruct((NPAD, C), jnp.float32),
    )(block_expert, nb_active, xs, gate_weight, up_weight, down_weight)


_SH_BT = 256


def _shared_body(x_ref, gw_ref, uw_ref, dw_ref, o_ref):
    xb = x_ref[...]
    g = lax.dot_general(xb, gw_ref[...], (((1,), (1,)), ((), ())),
                        preferred_element_type=jnp.float32)
    u = lax.dot_general(xb, uw_ref[...], (((1,), (1,)), ((), ())),
                        preferred_element_type=jnp.float32)
    h = g * jax.nn.sigmoid(g) * u
    o_ref[...] = lax.dot_general(h, dw_ref[...], (((1,), (1,)), ((), ())),
                                 preferred_element_type=jnp.float32)


def _run_shared(x_flat, shared_gate_w, shared_up_w, shared_down_w):
    return pl.pallas_call(
        _shared_body,
        grid=(S // _SH_BT,),
        in_specs=[
            pl.BlockSpec((_SH_BT, C), lambda t: (t, 0)),
            pl.BlockSpec((SH, C), lambda t: (0, 0)),
            pl.BlockSpec((SH, C), lambda t: (0, 0)),
            pl.BlockSpec((C, SH), lambda t: (0, 0)),
        ],
        out_specs=pl.BlockSpec((_SH_BT, C), lambda t: (t, 0)),
        out_shape=jax.ShapeDtypeStruct((S, C), jnp.float32),
    )(x_flat, shared_gate_w, shared_up_w, shared_down_w)


_C_TPW = S // NW
_C_TCH = 8
_C_NCH = _C_TPW // _C_TCH


def _combine_body(ys_hbm, pos_hbm, fwb_hbm, sh_hbm, out_hbm,
                  i0, i1, r0, r1, fwb_v, sh_v, o0, o1,
                  g0, g1, s0, s1):
    wid = lax.axis_index("s") * NC + lax.axis_index("c")
    idxs, rows, outs = (i0, i1), (r0, r1), (o0, o1)
    gsem, osem = (g0, g1), (s0, s1)

    def start_gather(c):
        tok0 = wid * _C_TPW + c * _C_TCH
        pltpu.sync_copy(pos_hbm.at[pl.ds(tok0 * K, _C_TCH * K)], idxs[c % 2])
        return pltpu.async_copy(ys_hbm.at[idxs[c % 2]], rows[c % 2],
                                gsem[c % 2])

    gd = [None] * _C_NCH
    od = [None] * _C_NCH
    gd[0] = start_gather(0)
    for c in range(_C_NCH):
        tok0 = wid * _C_TPW + c * _C_TCH
        if c + 1 < _C_NCH:
            gd[c + 1] = start_gather(c + 1)
        pltpu.sync_copy(fwb_hbm.at[pl.ds(tok0, _C_TCH)], fwb_v)
        pltpu.sync_copy(sh_hbm.at[pl.ds(tok0, _C_TCH)], sh_v)
        gd[c].wait()
        if c >= 2:
            od[c - 2].wait()
        rv = rows[c % 2]
        ov = outs[c % 2]
        for t in range(_C_TCH):
            fwj = [fwb_v[t, pl.ds(j * 16, 16)] for j in range(K)]

            def lanes(v, carry2, t=t, fwj=fwj, rv=rv, ov=ov):
                sl = pl.ds(v * L, L)
                acc = sh_v[t, sl]
                for j in range(K):
                    acc = acc + rv[t * K + j, sl].astype(jnp.float32) * fwj[j]
                ov[t, sl] = acc
                return carry2

            lax.fori_loop(0, C // L, lanes, 0)
        od[c] = pltpu.async_copy(ov, out_hbm.at[pl.ds(tok0, _C_TCH)],
                                 osem[c % 2])
    od[_C_NCH - 2].wait()
    od[_C_NCH - 1].wait()


def _run_combine(ys, pos_flat, fw_bc, shared_out):
    mesh = plsc.VectorSubcoreMesh(core_axis_name="c", subcore_axis_name="s", num_cores=NC, num_subcores=NS)
    return pl.kernel(
        _combine_body,
        out_type=jax.ShapeDtypeStruct((S, C), jnp.float32),
        mesh=mesh,
        scratch_types=[
            pltpu.VMEM((_C_TCH * K,), jnp.int32),
            pltpu.VMEM((_C_TCH * K,), jnp.int32),
            pltpu.VMEM((_C_TCH * K, C), jnp.float32),
            pltpu.VMEM((_C_TCH * K, C), jnp.float32),
            pltpu.VMEM((_C_TCH, K * 16), jnp.float32),
            pltpu.VMEM((_C_TCH, C), jnp.float32),
            pltpu.VMEM((_C_TCH, C), jnp.float32),
            pltpu.VMEM((_C_TCH, C), jnp.float32),
            pltpu.SemaphoreType.DMA,
            pltpu.SemaphoreType.DMA,
            pltpu.SemaphoreType.DMA,
            pltpu.SemaphoreType.DMA,
        ],
    )(ys, pos_flat, fw_bc, shared_out)


@jax.jit
def kernel(x, router_w, e_bias, gate_weight, up_weight, down_weight,
           shared_gate_w, shared_up_w, shared_down_w):
    x_flat = x.reshape(S, C)
    pos, fw_bc, x_bf16, block_expert, nb_active = _run_router(
        x_flat, router_w, e_bias)
    pos_flat = pos.reshape(SK)
    pos3 = pos.T.reshape(K, NW, _D_TPW).transpose(1, 0, 2)
    xs = _run_dispatch(x_bf16, pos3)
    shared_out = _run_shared(x_flat, shared_gate_w, shared_up_w, shared_down_w)
    ys = _run_experts(xs, gate_weight, up_weight, down_weight,
                      block_expert.reshape(NB), nb_active.reshape(1))
    out = _run_combine(ys, pos_flat, fw_bc, shared_out)
    return out.reshape(B, T, C)


# speedup vs baseline: 2.3352x; 1.0540x over previous
import functools
import jax
import jax.numpy as jnp
from jax import lax
from jax.experimental import pallas as pl
from jax.experimental.pallas import tpu as pltpu
from jax.experimental.pallas import tpu_sc as plsc

B, T, C = 1, 2048, 768
E, K, G, TG = 64, 8, 8, 4
H, SH = 384, 1536
S = B * T
SK = S * K
EPG = E // G
R = 128
NB = E + SK // R
NPAD = NB * R
NEG = -1e30

NC, NS, L = 2, 16, 16
NW = NC * NS


def _router_body(x_ref, rw_ref, eb_ref, pos_ref, fwb_ref, xpk_ref, be_ref, nba_ref):
    x = x_ref[...]
    rw = rw_ref[...]
    logits = lax.dot_general(x, rw, (((1,), (1,)), ((), ())),
                             preferred_element_type=jnp.float32)
    scores = jax.nn.sigmoid(logits)
    sb = scores + eb_ref[...]

    sb3 = sb.reshape(S, G, EPG)
    io3 = lax.broadcasted_iota(jnp.int32, (S, G, EPG), 2)
    m1 = jnp.max(sb3, axis=2)
    a1 = jnp.min(jnp.where(sb3 == m1[:, :, None], io3, EPG), axis=2)
    m2 = jnp.max(jnp.where(io3 == a1[:, :, None], NEG, sb3), axis=2)
    group_scores = m1 + m2

    iog = lax.broadcasted_iota(jnp.int32, (S, G), 1)
    gs = group_scores
    gmask = jnp.zeros((S, G), jnp.float32)
    for _ in range(TG):
        m = jnp.max(gs, axis=1)
        a = jnp.min(jnp.where(gs == m[:, None], iog, G), axis=1)
        sel = iog == a[:, None]
        gmask = jnp.where(sel, 1.0, gmask)
        gs = jnp.where(sel, NEG, gs)
    score_mask = jnp.repeat(gmask, EPG, axis=1)
    masked = jnp.where(score_mask == 0, NEG, sb)

    ioe = lax.broadcasted_iota(jnp.int32, (S, E), 1)
    topk = []
    fws = []
    for _ in range(K):
        m = jnp.max(masked, axis=1)
        a = jnp.min(jnp.where(masked == m[:, None], ioe, E), axis=1)
        sel = ioe == a[:, None]
        topk.append(a)
        fws.append(jnp.sum(jnp.where(sel, scores, 0.0), axis=1))
        masked = jnp.where(sel, NEG, masked)
    fw = jnp.stack(fws, axis=1)
    fw = fw / (jnp.sum(fw, axis=1, keepdims=True) + 1e-20)
    fwb_ref[...] = jnp.repeat(fw, 16, axis=1)
    xpk_ref[...] = pltpu.pack_elementwise(
        [x[:, :C // 2], x[:, C // 2:]], packed_dtype=jnp.bfloat16)

    hist = jnp.zeros((S, E), jnp.float32)
    for k in range(K):
        hist = hist + jnp.where(ioe == topk[k][:, None], 1.0, 0.0)
    ior = lax.broadcasted_iota(jnp.int32, (S, S), 0)
    ioc = lax.broadcasted_iota(jnp.int32, (S, S), 1)
    tri = jnp.where(ior > ioc, 1.0, 0.0)
    csum_excl = lax.dot_general(tri, hist, (((1,), (0,)), ((), ())),
                                preferred_element_type=jnp.float32)
    counts = jnp.sum(hist, axis=0, keepdims=True).astype(jnp.int32)
    pc = ((counts + (R - 1)) >> 7) << 7
    ie1 = lax.broadcasted_iota(jnp.int32, (E, E), 0)
    ie2 = lax.broadcasted_iota(jnp.int32, (E, E), 1)
    trie = jnp.where(ie1 < ie2, 1.0, 0.0)
    start = lax.dot_general(pc.astype(jnp.float32), trie,
                            (((1,), (0,)), ((), ())),
                            preferred_element_type=jnp.float32)

    slot_base = start + csum_excl
    pos_cols = []
    for k in range(K):
        sel = ioe == topk[k][:, None]
        pos_k = jnp.sum(jnp.where(sel, slot_base, 0.0), axis=1)
        pos_cols.append(pos_k.astype(jnp.int32))
    pos_ref[...] = jnp.stack(pos_cols, axis=1)

    start_i = start.astype(jnp.int32)
    iob = lax.broadcasted_iota(jnp.int32, (E, NB), 1)
    cmp = jnp.where(start_i.reshape(E, 1) <= iob * R, 1, 0)
    be = jnp.sum(cmp, axis=0, keepdims=True) - 1
    be_ref[...] = jnp.clip(be, 0, E - 1).astype(jnp.int32)
    total = jnp.sum(pc, axis=1, keepdims=True)
    nba_ref[...] = total >> 7


def _run_router(x_flat, router_w, e_bias):
    return pl.pallas_call(
        _router_body,
        out_shape=[
            jax.ShapeDtypeStruct((S, K), jnp.int32),
            jax.ShapeDtypeStruct((S, K * 16), jnp.float32),
            jax.ShapeDtypeStruct((S, C // 2), jnp.int32),
            jax.ShapeDtypeStruct((1, NB), jnp.int32),
            jax.ShapeDtypeStruct((1, 1), jnp.int32),
        ],
    )(x_flat, router_w, e_bias.reshape(1, E))


_D_TPW = S // NW


def _dispatch_body(x_hbm, pos3_hbm, xs_hbm, idx_v, buf, sem):
    wid = lax.axis_index("s") * NC + lax.axis_index("c")
    t0 = wid * _D_TPW
    pltpu.sync_copy(pos3_hbm.at[wid], idx_v)
    pltpu.sync_copy(x_hbm.at[pl.ds(t0, _D_TPW)], buf)
    cps = [pltpu.async_copy(buf, xs_hbm.at[idx_v.at[k]], sem)
           for k in range(K)]
    for cp in cps:
        cp.wait()


def _run_dispatch(x_bf16, pos3):
    mesh = plsc.VectorSubcoreMesh(core_axis_name="c", subcore_axis_name="s", num_cores=NC, num_subcores=NS)
    return pl.kernel(
        _dispatch_body,
        out_type=jax.ShapeDtypeStruct((NPAD, C // 2), jnp.int32),
        mesh=mesh,
        scratch_types=[
            pltpu.VMEM((K, _D_TPW), jnp.int32),
            pltpu.VMEM((_D_TPW, C // 2), jnp.int32),
            pltpu.SemaphoreType.DMA,
        ],
    )(x_bf16, pos3)


def _expert_body(be_ref, nba_ref, xs_ref, g_ref, u_ref, d_ref, ys_ref):
    b = pl.program_id(0)

    @pl.when(b < nba_ref[0])
    def _():
        xpk = xs_ref[...]
        xa = pltpu.unpack_elementwise(
            xpk, index=0, packed_dtype=jnp.bfloat16, unpacked_dtype=jnp.float32)
        xb_hi = pltpu.unpack_elementwise(
            xpk, index=1, packed_dtype=jnp.bfloat16, unpacked_dtype=jnp.float32)
        xb = jnp.concatenate([xa, xb_hi], axis=1).astype(jnp.bfloat16)
        gw = g_ref[0].astype(jnp.bfloat16)
        uw = u_ref[0].astype(jnp.bfloat16)
        dw = d_ref[0].astype(jnp.bfloat16)
        g = jnp.dot(xb, gw, preferred_element_type=jnp.float32)
        u = jnp.dot(xb, uw, preferred_element_type=jnp.float32)
        h = (g * jax.nn.sigmoid(g) * u).astype(jnp.bfloat16)
        y = jnp.dot(h, dw, preferred_element_type=jnp.float32)
        ys_ref[...] = pltpu.pack_elementwise(
            [y[:, :C // 2], y[:, C // 2:]], packed_dtype=jnp.bfloat16)


def _run_experts(xs, gate_weight, up_weight, down_weight,
                 block_expert, nb_active):
    def clamp(b, nba):
        return jnp.minimum(b, nba[0] - 1)

    grid_spec = pltpu.PrefetchScalarGridSpec(
        num_scalar_prefetch=2,
        grid=(NB,),
        in_specs=[
            pl.BlockSpec((R, C // 2), lambda b, be, nba: (clamp(b, nba), 0)),
            pl.BlockSpec((1, C, H), lambda b, be, nba: (be[clamp(b, nba)], 0, 0)),
            pl.BlockSpec((1, C, H), lambda b, be, nba: (be[clamp(b, nba)], 0, 0)),
            pl.BlockSpec((1, H, C), lambda b, be, nba: (be[clamp(b, nba)], 0, 0)),
        ],
        out_specs=pl.BlockSpec((R, C // 2), lambda b, be, nba: (clamp(b, nba), 0)),
    )
    return pl.pallas_call(
        _expert_body,
        grid_spec=grid_spec,
        out_shape=jax.ShapeDtypeStruct((NPAD, C // 2), jnp.int32),
    )(block_expert, nb_active, xs, gate_weight, up_weight, down_weight)


_SH_BT = 256


def _shared_body(x_ref, gw_ref, uw_ref, dw_ref, o_ref):
    xb = x_ref[...].astype(jnp.bfloat16)
    gw = gw_ref[...].astype(jnp.bfloat16)
    uw = uw_ref[...].astype(jnp.bfloat16)
    dw = dw_ref[...].astype(jnp.bfloat16)
    g = lax.dot_general(xb, gw, (((1,), (1,)), ((), ())),
                        preferred_element_type=jnp.float32)
    u = lax.dot_general(xb, uw, (((1,), (1,)), ((), ())),
                        preferred_element_type=jnp.float32)
    h = (g * jax.nn.sigmoid(g) * u).astype(jnp.bfloat16)
    o_ref[...] = lax.dot_general(h, dw, (((1,), (1,)), ((), ())),
                                 preferred_element_type=jnp.float32)


def _run_shared(x_flat, shared_gate_w, shared_up_w, shared_down_w):
    return pl.pallas_call(
        _shared_body,
        grid=(S // _SH_BT,),
        in_specs=[
            pl.BlockSpec((_SH_BT, C), lambda t: (t, 0)),
            pl.BlockSpec((SH, C), lambda t: (0, 0)),
            pl.BlockSpec((SH, C), lambda t: (0, 0)),
            pl.BlockSpec((C, SH), lambda t: (0, 0)),
        ],
        out_specs=pl.BlockSpec((_SH_BT, C), lambda t: (t, 0)),
        out_shape=jax.ShapeDtypeStruct((S, C), jnp.float32),
    )(x_flat, shared_gate_w, shared_up_w, shared_down_w)


_C_TPW = S // NW
_C_TCH = 8
_C_NCH = _C_TPW // _C_TCH


def _combine_body(ys_hbm, pos_hbm, fwb_hbm, sh_hbm, out_hbm,
                  i0, i1, r0, r1, fwb_v, sh_v, o0, o1,
                  g0, g1, s0, s1):
    wid = lax.axis_index("s") * NC + lax.axis_index("c")
    idxs, rows, outs = (i0, i1), (r0, r1), (o0, o1)
    gsem, osem = (g0, g1), (s0, s1)

    def start_gather(c):
        tok0 = wid * _C_TPW + c * _C_TCH
        pltpu.sync_copy(pos_hbm.at[pl.ds(tok0 * K, _C_TCH * K)], idxs[c % 2])
        return pltpu.async_copy(ys_hbm.at[idxs[c % 2]], rows[c % 2],
                                gsem[c % 2])

    gd = [None] * _C_NCH
    od = [None] * _C_NCH
    gd[0] = start_gather(0)
    for c in range(_C_NCH):
        tok0 = wid * _C_TPW + c * _C_TCH
        if c + 1 < _C_NCH:
            gd[c + 1] = start_gather(c + 1)
        pltpu.sync_copy(fwb_hbm.at[pl.ds(tok0, _C_TCH)], fwb_v)
        pltpu.sync_copy(sh_hbm.at[pl.ds(tok0, _C_TCH)], sh_v)
        gd[c].wait()
        if c >= 2:
            od[c - 2].wait()
        rv = rows[c % 2]
        ov = outs[c % 2]
        for t in range(_C_TCH):
            fwj = [fwb_v[t, pl.ds(j * 16, 16)] for j in range(K)]

            def lanes(v, carry2, t=t, fwj=fwj, rv=rv, ov=ov):
                sl = pl.ds(v * L, L)
                slh = pl.ds(C // 2 + v * L, L)
                acc_a = sh_v[t, sl]
                acc_b = sh_v[t, slh]
                for j in range(K):
                    pk = plsc.bitcast(rv[t * K + j, sl], jnp.bfloat16)
                    a, bb = plsc.unpack(pk, format=plsc.PackFormat.INTERLEAVED)
                    acc_a = acc_a + a * fwj[j]
                    acc_b = acc_b + bb * fwj[j]
                ov[t, sl] = acc_a
                ov[t, slh] = acc_b
                return carry2

            lax.fori_loop(0, C // (2 * L), lanes, 0)
        od[c] = pltpu.async_copy(ov, out_hbm.at[pl.ds(tok0, _C_TCH)],
                                 osem[c % 2])
    od[_C_NCH - 2].wait()
    od[_C_NCH - 1].wait()


def _run_combine(ys, pos_flat, fw_bc, shared_out):
    mesh = plsc.VectorSubcoreMesh(core_axis_name="c", subcore_axis_name="s", num_cores=NC, num_subcores=NS)
    return pl.kernel(
        _combine_body,
        compiler_params=pltpu.CompilerParams(needs_layout_passes=False),
        out_type=jax.ShapeDtypeStruct((S, C), jnp.float32),
        mesh=mesh,
        scratch_types=[
            pltpu.VMEM((_C_TCH * K,), jnp.int32),
            pltpu.VMEM((_C_TCH * K,), jnp.int32),
            pltpu.VMEM((_C_TCH * K, C // 2), jnp.int32),
            pltpu.VMEM((_C_TCH * K, C // 2), jnp.int32),
            pltpu.VMEM((_C_TCH, K * 16), jnp.float32),
            pltpu.VMEM((_C_TCH, C), jnp.float32),
            pltpu.VMEM((_C_TCH, C), jnp.float32),
            pltpu.VMEM((_C_TCH, C), jnp.float32),
            pltpu.SemaphoreType.DMA,
            pltpu.SemaphoreType.DMA,
            pltpu.SemaphoreType.DMA,
            pltpu.SemaphoreType.DMA,
        ],
    )(ys, pos_flat, fw_bc, shared_out)


@jax.jit
def kernel(x, router_w, e_bias, gate_weight, up_weight, down_weight,
           shared_gate_w, shared_up_w, shared_down_w):
    x_flat = x.reshape(S, C)
    pos, fw_bc, x_bf16, block_expert, nb_active = _run_router(
        x_flat, router_w, e_bias)
    pos_flat = pos.reshape(SK)
    pos3 = pos.T.reshape(K, NW, _D_TPW).transpose(1, 0, 2)
    xs = _run_dispatch(x_bf16, pos3)
    shared_out = _run_shared(x_flat, shared_gate_w, shared_up_w, shared_down_w)
    ys = _run_experts(xs, gate_weight, up_weight, down_weight,
                      block_expert.reshape(NB), nb_active.reshape(1))
    out = _run_combine(ys, pos_flat, fw_bc, shared_out)
    return out.reshape(B, T, C)


# speedup vs baseline: 2.3377x; 1.0011x over previous
import functools
import jax
import jax.numpy as jnp
from jax import lax
from jax.experimental import pallas as pl
from jax.experimental.pallas import tpu as pltpu
from jax.experimental.pallas import tpu_sc as plsc

B, T, C = 1, 2048, 768
E, K, G, TG = 64, 8, 8, 4
H, SH = 384, 1536
S = B * T
SK = S * K
EPG = E // G
R = 128
NB = E + SK // R
NPAD = NB * R
NEG = -1e30

NC, NS, L = 2, 16, 16
NW = NC * NS


def _router_body(x_ref, rw_ref, eb_ref, pos_ref, fwb_ref, xpk_ref, be_ref, nba_ref):
    x = x_ref[...]
    rw = rw_ref[...]
    logits = lax.dot_general(x, rw, (((1,), (1,)), ((), ())),
                             preferred_element_type=jnp.float32)
    scores = jax.nn.sigmoid(logits)
    sb = scores + eb_ref[...]

    sb3 = sb.reshape(S, G, EPG)
    io3 = lax.broadcasted_iota(jnp.int32, (S, G, EPG), 2)
    m1 = jnp.max(sb3, axis=2)
    a1 = jnp.min(jnp.where(sb3 == m1[:, :, None], io3, EPG), axis=2)
    m2 = jnp.max(jnp.where(io3 == a1[:, :, None], NEG, sb3), axis=2)
    group_scores = m1 + m2

    iog = lax.broadcasted_iota(jnp.int32, (S, G), 1)
    gs = group_scores
    gmask = jnp.zeros((S, G), jnp.float32)
    for _ in range(TG):
        m = jnp.max(gs, axis=1)
        a = jnp.min(jnp.where(gs == m[:, None], iog, G), axis=1)
        sel = iog == a[:, None]
        gmask = jnp.where(sel, 1.0, gmask)
        gs = jnp.where(sel, NEG, gs)
    score_mask = jnp.repeat(gmask, EPG, axis=1)
    masked = jnp.where(score_mask == 0, NEG, sb)

    ioe = lax.broadcasted_iota(jnp.int32, (S, E), 1)
    topk = []
    fws = []
    for _ in range(K):
        m = jnp.max(masked, axis=1)
        a = jnp.min(jnp.where(masked == m[:, None], ioe, E), axis=1)
        sel = ioe == a[:, None]
        topk.append(a)
        fws.append(jnp.sum(jnp.where(sel, scores, 0.0), axis=1))
        masked = jnp.where(sel, NEG, masked)
    fw = jnp.stack(fws, axis=1)
    fw = fw / (jnp.sum(fw, axis=1, keepdims=True) + 1e-20)
    fwb_ref[...] = jnp.repeat(fw, 16, axis=1)
    xpk_ref[...] = pltpu.pack_elementwise(
        [x[:, :C // 2], x[:, C // 2:]], packed_dtype=jnp.bfloat16)

    hist = jnp.zeros((S, E), jnp.float32)
    for k in range(K):
        hist = hist + jnp.where(ioe == topk[k][:, None], 1.0, 0.0)
    SB, BS = S // 16, 16
    h3 = hist.reshape(SB, BS, E)
    i1 = lax.broadcasted_iota(jnp.int32, (SB, BS, BS), 1)
    i2 = lax.broadcasted_iota(jnp.int32, (SB, BS, BS), 2)
    tri3 = jnp.where(i1 > i2, 1.0, 0.0)
    intra = lax.dot_general(tri3, h3, (((2,), (1,)), ((0,), (0,))),
                            preferred_element_type=jnp.float32)
    bsum = jnp.sum(h3, axis=1)
    ib1 = lax.broadcasted_iota(jnp.int32, (SB, SB), 0)
    ib2 = lax.broadcasted_iota(jnp.int32, (SB, SB), 1)
    trib = jnp.where(ib1 > ib2, 1.0, 0.0)
    boff = lax.dot_general(trib, bsum, (((1,), (0,)), ((), ())),
                           preferred_element_type=jnp.float32)
    csum_excl = (intra + boff[:, None, :]).reshape(S, E)
    counts = jnp.sum(hist, axis=0, keepdims=True).astype(jnp.int32)
    pc = ((counts + (R - 1)) >> 7) << 7
    ie1 = lax.broadcasted_iota(jnp.int32, (E, E), 0)
    ie2 = lax.broadcasted_iota(jnp.int32, (E, E), 1)
    trie = jnp.where(ie1 < ie2, 1.0, 0.0)
    start = lax.dot_general(pc.astype(jnp.float32), trie,
                            (((1,), (0,)), ((), ())),
                            preferred_element_type=jnp.float32)

    slot_base = start + csum_excl
    pos_cols = []
    for k in range(K):
        sel = ioe == topk[k][:, None]
        pos_k = jnp.sum(jnp.where(sel, slot_base, 0.0), axis=1)
        pos_cols.append(pos_k.astype(jnp.int32))
    pos_ref[...] = jnp.stack(pos_cols, axis=1)

    start_i = start.astype(jnp.int32)
    iob = lax.broadcasted_iota(jnp.int32, (E, NB), 1)
    cmp = jnp.where(start_i.reshape(E, 1) <= iob * R, 1, 0)
    be = jnp.sum(cmp, axis=0, keepdims=True) - 1
    be_ref[...] = jnp.clip(be, 0, E - 1).astype(jnp.int32)
    total = jnp.sum(pc, axis=1, keepdims=True)
    nba_ref[...] = total >> 7


def _run_router(x_flat, router_w, e_bias):
    return pl.pallas_call(
        _router_body,
        out_shape=[
            jax.ShapeDtypeStruct((S, K), jnp.int32),
            jax.ShapeDtypeStruct((S, K * 16), jnp.float32),
            jax.ShapeDtypeStruct((S, C // 2), jnp.int32),
            jax.ShapeDtypeStruct((1, NB), jnp.int32),
            jax.ShapeDtypeStruct((1, 1), jnp.int32),
        ],
    )(x_flat, router_w, e_bias.reshape(1, E))


_D_TPW = S // NW


def _dispatch_body(x_hbm, pos3_hbm, xs_hbm, idx_v, buf, sem):
    wid = lax.axis_index("s") * NC + lax.axis_index("c")
    t0 = wid * _D_TPW
    pltpu.sync_copy(pos3_hbm.at[wid], idx_v)
    pltpu.sync_copy(x_hbm.at[pl.ds(t0, _D_TPW)], buf)
    cps = [pltpu.async_copy(buf, xs_hbm.at[idx_v.at[k]], sem)
           for k in range(K)]
    for cp in cps:
        cp.wait()


def _run_dispatch(x_bf16, pos3):
    mesh = plsc.VectorSubcoreMesh(core_axis_name="c", subcore_axis_name="s", num_cores=NC, num_subcores=NS)
    return pl.kernel(
        _dispatch_body,
        out_type=jax.ShapeDtypeStruct((NPAD, C // 2), jnp.int32),
        mesh=mesh,
        scratch_types=[
            pltpu.VMEM((K, _D_TPW), jnp.int32),
            pltpu.VMEM((_D_TPW, C // 2), jnp.int32),
            pltpu.SemaphoreType.DMA,
        ],
    )(x_bf16, pos3)


def _expert_body(be_ref, nba_ref, xs_ref, g_ref, u_ref, d_ref, ys_ref):
    b = pl.program_id(0)

    @pl.when(b < nba_ref[0])
    def _():
        xpk = xs_ref[...]
        xa = pltpu.unpack_elementwise(
            xpk, index=0, packed_dtype=jnp.bfloat16, unpacked_dtype=jnp.float32)
        xb_hi = pltpu.unpack_elementwise(
            xpk, index=1, packed_dtype=jnp.bfloat16, unpacked_dtype=jnp.float32)
        xb = jnp.concatenate([xa, xb_hi], axis=1).astype(jnp.bfloat16)
        gw = g_ref[0].astype(jnp.bfloat16)
        uw = u_ref[0].astype(jnp.bfloat16)
        dw = d_ref[0].astype(jnp.bfloat16)
        g = jnp.dot(xb, gw, preferred_element_type=jnp.float32)
        u = jnp.dot(xb, uw, preferred_element_type=jnp.float32)
        h = (g * jax.nn.sigmoid(g) * u).astype(jnp.bfloat16)
        y = jnp.dot(h, dw, preferred_element_type=jnp.float32)
        ys_ref[...] = pltpu.pack_elementwise(
            [y[:, :C // 2], y[:, C // 2:]], packed_dtype=jnp.bfloat16)


def _run_experts(xs, gate_weight, up_weight, down_weight,
                 block_expert, nb_active):
    def clamp(b, nba):
        return jnp.minimum(b, nba[0] - 1)

    grid_spec = pltpu.PrefetchScalarGridSpec(
        num_scalar_prefetch=2,
        grid=(NB,),
        in_specs=[
            pl.BlockSpec((R, C // 2), lambda b, be, nba: (clamp(b, nba), 0)),
            pl.BlockSpec((1, C, H), lambda b, be, nba: (be[clamp(b, nba)], 0, 0)),
            pl.BlockSpec((1, C, H), lambda b, be, nba: (be[clamp(b, nba)], 0, 0)),
            pl.BlockSpec((1, H, C), lambda b, be, nba: (be[clamp(b, nba)], 0, 0)),
        ],
        out_specs=pl.BlockSpec((R, C // 2), lambda b, be, nba: (clamp(b, nba), 0)),
    )
    return pl.pallas_call(
        _expert_body,
        grid_spec=grid_spec,
        out_shape=jax.ShapeDtypeStruct((NPAD, C // 2), jnp.int32),
    )(block_expert, nb_active, xs, gate_weight, up_weight, down_weight)


_SH_BT = 256


def _shared_body(x_ref, gw_ref, uw_ref, dw_ref, o_ref):
    xb = x_ref[...].astype(jnp.bfloat16)
    gw = gw_ref[...].astype(jnp.bfloat16)
    uw = uw_ref[...].astype(jnp.bfloat16)
    dw = dw_ref[...].astype(jnp.bfloat16)
    g = lax.dot_general(xb, gw, (((1,), (1,)), ((), ())),
                        preferred_element_type=jnp.float32)
    u = lax.dot_general(xb, uw, (((1,), (1,)), ((), ())),
                        preferred_element_type=jnp.float32)
    h = (g * jax.nn.sigmoid(g) * u).astype(jnp.bfloat16)
    o_ref[...] = lax.dot_general(h, dw, (((1,), (1,)), ((), ())),
                                 preferred_element_type=jnp.float32)


def _run_shared(x_flat, shared_gate_w, shared_up_w, shared_down_w):
    return pl.pallas_call(
        _shared_body,
        grid=(S // _SH_BT,),
        in_specs=[
            pl.BlockSpec((_SH_BT, C), lambda t: (t, 0)),
            pl.BlockSpec((SH, C), lambda t: (0, 0)),
            pl.BlockSpec((SH, C), lambda t: (0, 0)),
            pl.BlockSpec((C, SH), lambda t: (0, 0)),
        ],
        out_specs=pl.BlockSpec((_SH_BT, C), lambda t: (t, 0)),
        out_shape=jax.ShapeDtypeStruct((S, C), jnp.float32),
    )(x_flat, shared_gate_w, shared_up_w, shared_down_w)


_C_TPW = S // NW
_C_TCH = 8
_C_NCH = _C_TPW // _C_TCH


def _combine_body(ys_hbm, pos_hbm, fwb_hbm, sh_hbm, out_hbm,
                  i0, i1, r0, r1, fwb_v, sh_v, o0, o1,
                  g0, g1, s0, s1):
    wid = lax.axis_index("s") * NC + lax.axis_index("c")
    idxs, rows, outs = (i0, i1), (r0, r1), (o0, o1)
    gsem, osem = (g0, g1), (s0, s1)

    def start_gather(c):
        tok0 = wid * _C_TPW + c * _C_TCH
        pltpu.sync_copy(pos_hbm.at[pl.ds(tok0 * K, _C_TCH * K)], idxs[c % 2])
        return pltpu.async_copy(ys_hbm.at[idxs[c % 2]], rows[c % 2],
                                gsem[c % 2])

    gd = [None] * _C_NCH
    od = [None] * _C_NCH
    gd[0] = start_gather(0)
    for c in range(_C_NCH):
        tok0 = wid * _C_TPW + c * _C_TCH
        if c + 1 < _C_NCH:
            gd[c + 1] = start_gather(c + 1)
        pltpu.sync_copy(fwb_hbm.at[pl.ds(tok0, _C_TCH)], fwb_v)
        pltpu.sync_copy(sh_hbm.at[pl.ds(tok0, _C_TCH)], sh_v)
        gd[c].wait()
        if c >= 2:
            od[c - 2].wait()
        rv = rows[c % 2]
        ov = outs[c % 2]
        for t in range(_C_TCH):
            fwj = [fwb_v[t, pl.ds(j * 16, 16)] for j in range(K)]

            def lanes(v, carry2, t=t, fwj=fwj, rv=rv, ov=ov):
                sl = pl.ds(v * L, L)
                slh = pl.ds(C // 2 + v * L, L)
                acc_a = sh_v[t, sl]
                acc_b = sh_v[t, slh]
                for j in range(K):
                    pk = plsc.bitcast(rv[t * K + j, sl], jnp.bfloat16)
                    a, bb = plsc.unpack(pk, format=plsc.PackFormat.INTERLEAVED)
                    acc_a = acc_a + a * fwj[j]
                    acc_b = acc_b + bb * fwj[j]
                ov[t, sl] = acc_a
                ov[t, slh] = acc_b
                return carry2

            lax.fori_loop(0, C // (2 * L), lanes, 0)
        od[c] = pltpu.async_copy(ov, out_hbm.at[pl.ds(tok0, _C_TCH)],
                                 osem[c % 2])
    od[_C_NCH - 2].wait()
    od[_C_NCH - 1].wait()


def _run_combine(ys, pos_flat, fw_bc, shared_out):
    mesh = plsc.VectorSubcoreMesh(core_axis_name="c", subcore_axis_name="s", num_cores=NC, num_subcores=NS)
    return pl.kernel(
        _combine_body,
        compiler_params=pltpu.CompilerParams(needs_layout_passes=False),
        out_type=jax.ShapeDtypeStruct((S, C), jnp.float32),
        mesh=mesh,
        scratch_types=[
            pltpu.VMEM((_C_TCH * K,), jnp.int32),
            pltpu.VMEM((_C_TCH * K,), jnp.int32),
            pltpu.VMEM((_C_TCH * K, C // 2), jnp.int32),
            pltpu.VMEM((_C_TCH * K, C // 2), jnp.int32),
            pltpu.VMEM((_C_TCH, K * 16), jnp.float32),
            pltpu.VMEM((_C_TCH, C), jnp.float32),
            pltpu.VMEM((_C_TCH, C), jnp.float32),
            pltpu.VMEM((_C_TCH, C), jnp.float32),
            pltpu.SemaphoreType.DMA,
            pltpu.SemaphoreType.DMA,
            pltpu.SemaphoreType.DMA,
            pltpu.SemaphoreType.DMA,
        ],
    )(ys, pos_flat, fw_bc, shared_out)


@jax.jit
def kernel(x, router_w, e_bias, gate_weight, up_weight, down_weight,
           shared_gate_w, shared_up_w, shared_down_w):
    x_flat = x.reshape(S, C)
    pos, fw_bc, x_bf16, block_expert, nb_active = _run_router(
        x_flat, router_w, e_bias)
    pos_flat = pos.reshape(SK)
    pos3 = pos.T.reshape(K, NW, _D_TPW).transpose(1, 0, 2)
    xs = _run_dispatch(x_bf16, pos3)
    shared_out = _run_shared(x_flat, shared_gate_w, shared_up_w, shared_down_w)
    ys = _run_experts(xs, gate_weight, up_weight, down_weight,
                      block_expert.reshape(NB), nb_active.reshape(1))
    out = _run_combine(ys, pos_flat, fw_bc, shared_out)
    return out.reshape(B, T, C)


# speedup vs baseline: 2.3435x; 1.0025x over previous
import functools
import jax
import jax.numpy as jnp
from jax import lax
from jax.experimental import pallas as pl
from jax.experimental.pallas import tpu as pltpu
from jax.experimental.pallas import tpu_sc as plsc

B, T, C = 1, 2048, 768
E, K, G, TG = 64, 8, 8, 4
H, SH = 384, 1536
S = B * T
SK = S * K
EPG = E // G
R = 128
NB = E + SK // R
NPAD = NB * R
NEG = -1e30

NC, NS, L = 2, 16, 16
NW = NC * NS


def _router_body(x_ref, rw_ref, eb_ref, pos_ref, fwb_ref, xpk_ref, be_ref, nba_ref):
    x = x_ref[...]
    rw = rw_ref[...]
    logits = lax.dot_general(x, rw, (((1,), (1,)), ((), ())),
                             preferred_element_type=jnp.float32)
    scores = jax.nn.sigmoid(logits)
    sb = scores + eb_ref[...]

    sb3 = sb.reshape(S, G, EPG)
    io3 = lax.broadcasted_iota(jnp.int32, (S, G, EPG), 2)
    m1 = jnp.max(sb3, axis=2)
    a1 = jnp.min(jnp.where(sb3 == m1[:, :, None], io3, EPG), axis=2)
    m2 = jnp.max(jnp.where(io3 == a1[:, :, None], NEG, sb3), axis=2)
    group_scores = m1 + m2

    iog = lax.broadcasted_iota(jnp.int32, (S, G), 1)
    gs = group_scores
    gmask = jnp.zeros((S, G), jnp.float32)
    for _ in range(TG):
        m = jnp.max(gs, axis=1)
        a = jnp.min(jnp.where(gs == m[:, None], iog, G), axis=1)
        sel = iog == a[:, None]
        gmask = jnp.where(sel, 1.0, gmask)
        gs = jnp.where(sel, NEG, gs)
    score_mask = jnp.repeat(gmask, EPG, axis=1)
    masked = jnp.where(score_mask == 0, NEG, sb)

    ioe = lax.broadcasted_iota(jnp.int32, (S, E), 1)
    topk = []
    fws = []
    for _ in range(K):
        m = jnp.max(masked, axis=1)
        a = jnp.min(jnp.where(masked == m[:, None], ioe, E), axis=1)
        sel = ioe == a[:, None]
        topk.append(a)
        fws.append(jnp.sum(jnp.where(sel, scores, 0.0), axis=1))
        masked = jnp.where(sel, NEG, masked)
    fw = jnp.stack(fws, axis=1)
    fw = fw / (jnp.sum(fw, axis=1, keepdims=True) + 1e-20)
    fwb_ref[...] = jnp.repeat(fw, 16, axis=1)
    xpk_ref[...] = pltpu.pack_elementwise(
        [x[:, :C // 2], x[:, C // 2:]], packed_dtype=jnp.bfloat16)

    hist = jnp.zeros((S, E), jnp.float32)
    for k in range(K):
        hist = hist + jnp.where(ioe == topk[k][:, None], 1.0, 0.0)
    SB, BS = S // 16, 16
    h3 = hist.reshape(SB, BS, E)
    i1 = lax.broadcasted_iota(jnp.int32, (SB, BS, BS), 1)
    i2 = lax.broadcasted_iota(jnp.int32, (SB, BS, BS), 2)
    tri3 = jnp.where(i1 > i2, 1.0, 0.0)
    intra = lax.dot_general(tri3, h3, (((2,), (1,)), ((0,), (0,))),
                            preferred_element_type=jnp.float32)
    bsum = jnp.sum(h3, axis=1)
    ib1 = lax.broadcasted_iota(jnp.int32, (SB, SB), 0)
    ib2 = lax.broadcasted_iota(jnp.int32, (SB, SB), 1)
    trib = jnp.where(ib1 > ib2, 1.0, 0.0)
    boff = lax.dot_general(trib, bsum, (((1,), (0,)), ((), ())),
                           preferred_element_type=jnp.float32)
    csum_excl = (intra + boff[:, None, :]).reshape(S, E)
    counts = jnp.sum(hist, axis=0, keepdims=True).astype(jnp.int32)
    pc = ((counts + (R - 1)) >> 7) << 7
    ie1 = lax.broadcasted_iota(jnp.int32, (E, E), 0)
    ie2 = lax.broadcasted_iota(jnp.int32, (E, E), 1)
    trie = jnp.where(ie1 < ie2, 1.0, 0.0)
    start = lax.dot_general(pc.astype(jnp.float32), trie,
                            (((1,), (0,)), ((), ())),
                            preferred_element_type=jnp.float32)

    slot_base = start + csum_excl
    pos_cols = []
    for k in range(K):
        sel = ioe == topk[k][:, None]
        pos_k = jnp.sum(jnp.where(sel, slot_base, 0.0), axis=1)
        pos_cols.append(pos_k.astype(jnp.int32))
    pos_ref[...] = jnp.stack(pos_cols, axis=1)

    start_i = start.astype(jnp.int32)
    iob = lax.broadcasted_iota(jnp.int32, (E, NB), 1)
    cmp = jnp.where(start_i.reshape(E, 1) <= iob * R, 1, 0)
    be = jnp.sum(cmp, axis=0, keepdims=True) - 1
    be_ref[...] = jnp.clip(be, 0, E - 1).astype(jnp.int32)
    total = jnp.sum(pc, axis=1, keepdims=True)
    nba_ref[...] = total >> 7


def _run_router(x_flat, router_w, e_bias):
    return pl.pallas_call(
        _router_body,
        out_shape=[
            jax.ShapeDtypeStruct((S, K), jnp.int32),
            jax.ShapeDtypeStruct((S, K * 16), jnp.float32),
            jax.ShapeDtypeStruct((S, C // 2), jnp.int32),
            jax.ShapeDtypeStruct((1, NB), jnp.int32),
            jax.ShapeDtypeStruct((1, 1), jnp.int32),
        ],
    )(x_flat, router_w, e_bias.reshape(1, E))


_D_TPW = S // NW


def _dispatch_body(x_hbm, pos3_hbm, xs_hbm, idx_v, buf, sem):
    wid = lax.axis_index("s") * NC + lax.axis_index("c")
    t0 = wid * _D_TPW
    pltpu.sync_copy(pos3_hbm.at[wid], idx_v)
    pltpu.sync_copy(x_hbm.at[pl.ds(t0, _D_TPW)], buf)
    cps = [pltpu.async_copy(buf, xs_hbm.at[idx_v.at[k]], sem)
           for k in range(K)]
    for cp in cps:
        cp.wait()


def _run_dispatch(x_bf16, pos3):
    mesh = plsc.VectorSubcoreMesh(core_axis_name="c", subcore_axis_name="s", num_cores=NC, num_subcores=NS)
    return pl.kernel(
        _dispatch_body,
        out_type=jax.ShapeDtypeStruct((NPAD, C // 2), jnp.int32),
        mesh=mesh,
        scratch_types=[
            pltpu.VMEM((K, _D_TPW), jnp.int32),
            pltpu.VMEM((_D_TPW, C // 2), jnp.int32),
            pltpu.SemaphoreType.DMA,
        ],
    )(x_bf16, pos3)


def _expert_body(be_ref, nba_ref, xs_ref, g_ref, u_ref, d_ref, ys_ref):
    b = pl.program_id(0)

    @pl.when(b < nba_ref[0])
    def _():
        xpk = xs_ref[...]
        xa = pltpu.unpack_elementwise(
            xpk, index=0, packed_dtype=jnp.bfloat16, unpacked_dtype=jnp.float32)
        xb_hi = pltpu.unpack_elementwise(
            xpk, index=1, packed_dtype=jnp.bfloat16, unpacked_dtype=jnp.float32)
        xb = jnp.concatenate([xa, xb_hi], axis=1).astype(jnp.bfloat16)
        gw = g_ref[0].astype(jnp.bfloat16)
        uw = u_ref[0].astype(jnp.bfloat16)
        dw = d_ref[0].astype(jnp.bfloat16)
        g = jnp.dot(xb, gw, preferred_element_type=jnp.float32)
        u = jnp.dot(xb, uw, preferred_element_type=jnp.float32)
        h = (g * jax.nn.sigmoid(g) * u).astype(jnp.bfloat16)
        y = jnp.dot(h, dw, preferred_element_type=jnp.float32)
        ys_ref[...] = pltpu.pack_elementwise(
            [y[:, :C // 2], y[:, C // 2:]], packed_dtype=jnp.bfloat16)


def _run_experts(xs, gate_weight, up_weight, down_weight,
                 block_expert, nb_active):
    def clamp(b, nba):
        return jnp.minimum(b, nba[0] - 1)

    grid_spec = pltpu.PrefetchScalarGridSpec(
        num_scalar_prefetch=2,
        grid=(NB,),
        in_specs=[
            pl.BlockSpec((R, C // 2), lambda b, be, nba: (clamp(b, nba), 0)),
            pl.BlockSpec((1, C, H), lambda b, be, nba: (be[clamp(b, nba)], 0, 0)),
            pl.BlockSpec((1, C, H), lambda b, be, nba: (be[clamp(b, nba)], 0, 0)),
            pl.BlockSpec((1, H, C), lambda b, be, nba: (be[clamp(b, nba)], 0, 0)),
        ],
        out_specs=pl.BlockSpec((R, C // 2), lambda b, be, nba: (clamp(b, nba), 0)),
    )
    return pl.pallas_call(
        _expert_body,
        grid_spec=grid_spec,
        out_shape=jax.ShapeDtypeStruct((NPAD, C // 2), jnp.int32),
    )(block_expert, nb_active, xs, gate_weight, up_weight, down_weight)


_SH_BT = 1024


def _shared_body(x_ref, gw_ref, uw_ref, dw_ref, o_ref):
    xb = x_ref[...].astype(jnp.bfloat16)
    gw = gw_ref[...].astype(jnp.bfloat16)
    uw = uw_ref[...].astype(jnp.bfloat16)
    dw = dw_ref[...].astype(jnp.bfloat16)
    g = lax.dot_general(xb, gw, (((1,), (1,)), ((), ())),
                        preferred_element_type=jnp.float32)
    u = lax.dot_general(xb, uw, (((1,), (1,)), ((), ())),
                        preferred_element_type=jnp.float32)
    h = (g * jax.nn.sigmoid(g) * u).astype(jnp.bfloat16)
    o_ref[...] = lax.dot_general(h, dw, (((1,), (1,)), ((), ())),
                                 preferred_element_type=jnp.float32)


def _run_shared(x_flat, shared_gate_w, shared_up_w, shared_down_w):
    return pl.pallas_call(
        _shared_body,
        grid=(S // _SH_BT,),
        in_specs=[
            pl.BlockSpec((_SH_BT, C), lambda t: (t, 0)),
            pl.BlockSpec((SH, C), lambda t: (0, 0)),
            pl.BlockSpec((SH, C), lambda t: (0, 0)),
            pl.BlockSpec((C, SH), lambda t: (0, 0)),
        ],
        out_specs=pl.BlockSpec((_SH_BT, C), lambda t: (t, 0)),
        out_shape=jax.ShapeDtypeStruct((S, C), jnp.float32),
    )(x_flat, shared_gate_w, shared_up_w, shared_down_w)


_C_TPW = S // NW
_C_TCH = 8
_C_NCH = _C_TPW // _C_TCH


def _combine_body(ys_hbm, pos_hbm, fwb_hbm, sh_hbm, out_hbm,
                  i0, i1, r0, r1, fwb_v, sh_v, o0, o1,
                  g0, g1, s0, s1):
    wid = lax.axis_index("s") * NC + lax.axis_index("c")
    idxs, rows, outs = (i0, i1), (r0, r1), (o0, o1)
    gsem, osem = (g0, g1), (s0, s1)

    def start_gather(c):
        tok0 = wid * _C_TPW + c * _C_TCH
        pltpu.sync_copy(pos_hbm.at[pl.ds(tok0 * K, _C_TCH * K)], idxs[c % 2])
        return pltpu.async_copy(ys_hbm.at[idxs[c % 2]], rows[c % 2],
                                gsem[c % 2])

    gd = [None] * _C_NCH
    od = [None] * _C_NCH
    gd[0] = start_gather(0)
    for c in range(_C_NCH):
        tok0 = wid * _C_TPW + c * _C_TCH
        if c + 1 < _C_NCH:
            gd[c + 1] = start_gather(c + 1)
        pltpu.sync_copy(fwb_hbm.at[pl.ds(tok0, _C_TCH)], fwb_v)
        pltpu.sync_copy(sh_hbm.at[pl.ds(tok0, _C_TCH)], sh_v)
        gd[c].wait()
        if c >= 2:
            od[c - 2].wait()
        rv = rows[c % 2]
        ov = outs[c % 2]
        for t in range(_C_TCH):
            fwj = [fwb_v[t, pl.ds(j * 16, 16)] for j in range(K)]

            def lanes(v, carry2, t=t, fwj=fwj, rv=rv, ov=ov):
                sl = pl.ds(v * L, L)
                slh = pl.ds(C // 2 + v * L, L)
                acc_a = sh_v[t, sl]
                acc_b = sh_v[t, slh]
                for j in range(K):
                    pk = plsc.bitcast(rv[t * K + j, sl], jnp.bfloat16)
                    a, bb = plsc.unpack(pk, format=plsc.PackFormat.INTERLEAVED)
                    acc_a = acc_a + a * fwj[j]
                    acc_b = acc_b + bb * fwj[j]
                ov[t, sl] = acc_a
                ov[t, slh] = acc_b
                return carry2

            lax.fori_loop(0, C // (2 * L), lanes, 0)
        od[c] = pltpu.async_copy(ov, out_hbm.at[pl.ds(tok0, _C_TCH)],
                                 osem[c % 2])
    od[_C_NCH - 2].wait()
    od[_C_NCH - 1].wait()


def _run_combine(ys, pos_flat, fw_bc, shared_out):
    mesh = plsc.VectorSubcoreMesh(core_axis_name="c", subcore_axis_name="s", num_cores=NC, num_subcores=NS)
    return pl.kernel(
        _combine_body,
        compiler_params=pltpu.CompilerParams(needs_layout_passes=False),
        out_type=jax.ShapeDtypeStruct((S, C), jnp.float32),
        mesh=mesh,
        scratch_types=[
            pltpu.VMEM((_C_TCH * K,), jnp.int32),
            pltpu.VMEM((_C_TCH * K,), jnp.int32),
            pltpu.VMEM((_C_TCH * K, C // 2), jnp.int32),
            pltpu.VMEM((_C_TCH * K, C // 2), jnp.int32),
            pltpu.VMEM((_C_TCH, K * 16), jnp.float32),
            pltpu.VMEM((_C_TCH, C), jnp.float32),
            pltpu.VMEM((_C_TCH, C), jnp.float32),
            pltpu.VMEM((_C_TCH, C), jnp.float32),
            pltpu.SemaphoreType.DMA,
            pltpu.SemaphoreType.DMA,
            pltpu.SemaphoreType.DMA,
            pltpu.SemaphoreType.DMA,
        ],
    )(ys, pos_flat, fw_bc, shared_out)


@jax.jit
def kernel(x, router_w, e_bias, gate_weight, up_weight, down_weight,
           shared_gate_w, shared_up_w, shared_down_w):
    x_flat = x.reshape(S, C)
    pos, fw_bc, x_bf16, block_expert, nb_active = _run_router(
        x_flat, router_w, e_bias)
    pos_flat = pos.reshape(SK)
    pos3 = pos.T.reshape(K, NW, _D_TPW).transpose(1, 0, 2)
    xs = _run_dispatch(x_bf16, pos3)
    shared_out = _run_shared(x_flat, shared_gate_w, shared_up_w, shared_down_w)
    ys = _run_experts(xs, gate_weight, up_weight, down_weight,
                      block_expert.reshape(NB), nb_active.reshape(1))
    out = _run_combine(ys, pos_flat, fw_bc, shared_out)
    return out.reshape(B, T, C)


# speedup vs baseline: 2.3498x; 1.0027x over previous
import functools
import jax
import jax.numpy as jnp
from jax import lax
from jax.experimental import pallas as pl
from jax.experimental.pallas import tpu as pltpu
from jax.experimental.pallas import tpu_sc as plsc

B, T, C = 1, 2048, 768
E, K, G, TG = 64, 8, 8, 4
H, SH = 384, 1536
S = B * T
SK = S * K
EPG = E // G
R = 128
NB = E + SK // R
NPAD = NB * R
NEG = -1e30

NC, NS, L = 2, 16, 16
NW = NC * NS


def _router_body(x_ref, rw_ref, eb_ref, pos_ref, fwb_ref, xpk_ref, be_ref, nba_ref):
    x = x_ref[...]
    rw = rw_ref[...]
    logits = lax.dot_general(x, rw, (((1,), (1,)), ((), ())),
                             preferred_element_type=jnp.float32)
    scores = jax.nn.sigmoid(logits)
    sb = scores + eb_ref[...]

    sb3 = sb.reshape(S, G, EPG)
    io3 = lax.broadcasted_iota(jnp.int32, (S, G, EPG), 2)
    m1 = jnp.max(sb3, axis=2)
    a1 = jnp.min(jnp.where(sb3 == m1[:, :, None], io3, EPG), axis=2)
    m2 = jnp.max(jnp.where(io3 == a1[:, :, None], NEG, sb3), axis=2)
    group_scores = m1 + m2

    iog = lax.broadcasted_iota(jnp.int32, (S, G), 1)
    gs = group_scores
    gmask = jnp.zeros((S, G), jnp.float32)
    for _ in range(TG):
        m = jnp.max(gs, axis=1)
        a = jnp.min(jnp.where(gs == m[:, None], iog, G), axis=1)
        sel = iog == a[:, None]
        gmask = jnp.where(sel, 1.0, gmask)
        gs = jnp.where(sel, NEG, gs)
    score_mask = jnp.repeat(gmask, EPG, axis=1)
    masked = jnp.where(score_mask == 0, NEG, sb)

    ioe = lax.broadcasted_iota(jnp.int32, (S, E), 1)
    topk = []
    fws = []
    for _ in range(K):
        m = jnp.max(masked, axis=1)
        a = jnp.min(jnp.where(masked == m[:, None], ioe, E), axis=1)
        sel = ioe == a[:, None]
        topk.append(a)
        fws.append(jnp.sum(jnp.where(sel, scores, 0.0), axis=1))
        masked = jnp.where(sel, NEG, masked)
    fw = jnp.stack(fws, axis=1)
    fw = fw / (jnp.sum(fw, axis=1, keepdims=True) + 1e-20)
    fwb_ref[...] = jnp.repeat(fw, 16, axis=1)
    xpk_ref[...] = pltpu.pack_elementwise(
        [x[:, :C // 2], x[:, C // 2:]], packed_dtype=jnp.bfloat16)

    hist = jnp.zeros((S, E), jnp.float32)
    for k in range(K):
        hist = hist + jnp.where(ioe == topk[k][:, None], 1.0, 0.0)
    SB, BS = S // 16, 16
    h3 = hist.reshape(SB, BS, E)
    i1 = lax.broadcasted_iota(jnp.int32, (SB, BS, BS), 1)
    i2 = lax.broadcasted_iota(jnp.int32, (SB, BS, BS), 2)
    tri3 = jnp.where(i1 > i2, 1.0, 0.0)
    intra = lax.dot_general(tri3, h3, (((2,), (1,)), ((0,), (0,))),
                            preferred_element_type=jnp.float32)
    bsum = jnp.sum(h3, axis=1)
    ib1 = lax.broadcasted_iota(jnp.int32, (SB, SB), 0)
    ib2 = lax.broadcasted_iota(jnp.int32, (SB, SB), 1)
    trib = jnp.where(ib1 > ib2, 1.0, 0.0)
    boff = lax.dot_general(trib, bsum, (((1,), (0,)), ((), ())),
                           preferred_element_type=jnp.float32)
    csum_excl = (intra + boff[:, None, :]).reshape(S, E)
    counts = jnp.sum(hist, axis=0, keepdims=True).astype(jnp.int32)
    pc = ((counts + (R - 1)) >> 7) << 7
    ie1 = lax.broadcasted_iota(jnp.int32, (E, E), 0)
    ie2 = lax.broadcasted_iota(jnp.int32, (E, E), 1)
    trie = jnp.where(ie1 < ie2, 1.0, 0.0)
    start = lax.dot_general(pc.astype(jnp.float32), trie,
                            (((1,), (0,)), ((), ())),
                            preferred_element_type=jnp.float32)

    slot_base = start + csum_excl
    pos_cols = []
    for k in range(K):
        sel = ioe == topk[k][:, None]
        pos_k = jnp.sum(jnp.where(sel, slot_base, 0.0), axis=1)
        pos_cols.append(pos_k.astype(jnp.int32))
    pos_ref[...] = jnp.stack(pos_cols, axis=1)

    start_i = start.astype(jnp.int32)
    iob = lax.broadcasted_iota(jnp.int32, (E, NB), 1)
    cmp = jnp.where(start_i.reshape(E, 1) <= iob * R, 1, 0)
    be = jnp.sum(cmp, axis=0, keepdims=True) - 1
    be_ref[...] = jnp.clip(be, 0, E - 1).astype(jnp.int32)
    total = jnp.sum(pc, axis=1, keepdims=True)
    nba_ref[...] = total >> 7


def _run_router(x_flat, router_w, e_bias):
    return pl.pallas_call(
        _router_body,
        out_shape=[
            jax.ShapeDtypeStruct((S, K), jnp.int32),
            jax.ShapeDtypeStruct((S, K * 16), jnp.float32),
            jax.ShapeDtypeStruct((S, C // 2), jnp.int32),
            jax.ShapeDtypeStruct((1, NB), jnp.int32),
            jax.ShapeDtypeStruct((1, 1), jnp.int32),
        ],
    )(x_flat, router_w, e_bias.reshape(1, E))


_D_TPW = S // NW


def _dispatch_body(x_hbm, pos3_hbm, xs_hbm, idx_v, buf, sem):
    wid = lax.axis_index("s") * NC + lax.axis_index("c")
    t0 = wid * _D_TPW
    pltpu.sync_copy(pos3_hbm.at[wid], idx_v)
    pltpu.sync_copy(x_hbm.at[pl.ds(t0, _D_TPW)], buf)
    cps = [pltpu.async_copy(buf, xs_hbm.at[idx_v.at[k]], sem)
           for k in range(K)]
    for cp in cps:
        cp.wait()


def _run_dispatch(x_bf16, pos3):
    mesh = plsc.VectorSubcoreMesh(core_axis_name="c", subcore_axis_name="s", num_cores=NC, num_subcores=NS)
    return pl.kernel(
        _dispatch_body,
        out_type=jax.ShapeDtypeStruct((NPAD, C // 2), jnp.int32),
        mesh=mesh,
        scratch_types=[
            pltpu.VMEM((K, _D_TPW), jnp.int32),
            pltpu.VMEM((_D_TPW, C // 2), jnp.int32),
            pltpu.SemaphoreType.DMA,
        ],
    )(x_bf16, pos3)


def _expert_body(be_ref, nba_ref, xs_ref, g_ref, u_ref, d_ref, ys_ref):
    b = pl.program_id(0)

    @pl.when(b < nba_ref[0])
    def _():
        xpk = xs_ref[...]
        xa = pltpu.unpack_elementwise(
            xpk, index=0, packed_dtype=jnp.bfloat16, unpacked_dtype=jnp.float32)
        xb_hi = pltpu.unpack_elementwise(
            xpk, index=1, packed_dtype=jnp.bfloat16, unpacked_dtype=jnp.float32)
        xb = jnp.concatenate([xa, xb_hi], axis=1).astype(jnp.bfloat16)
        gw = g_ref[0].astype(jnp.bfloat16)
        uw = u_ref[0].astype(jnp.bfloat16)
        dw = d_ref[0].astype(jnp.bfloat16)
        g = jnp.dot(xb, gw, preferred_element_type=jnp.float32)
        u = jnp.dot(xb, uw, preferred_element_type=jnp.float32)
        h = (g * jax.nn.sigmoid(g) * u).astype(jnp.bfloat16)
        y = jnp.dot(h, dw, preferred_element_type=jnp.float32)
        ys_ref[...] = pltpu.pack_elementwise(
            [y[:, :C // 2], y[:, C // 2:]], packed_dtype=jnp.bfloat16)


def _run_experts(xs, gate_weight, up_weight, down_weight,
                 block_expert, nb_active):
    def clamp(b, nba):
        return jnp.minimum(b, nba[0] - 1)

    grid_spec = pltpu.PrefetchScalarGridSpec(
        num_scalar_prefetch=2,
        grid=(NB,),
        in_specs=[
            pl.BlockSpec((R, C // 2), lambda b, be, nba: (clamp(b, nba), 0)),
            pl.BlockSpec((1, C, H), lambda b, be, nba: (be[clamp(b, nba)], 0, 0)),
            pl.BlockSpec((1, C, H), lambda b, be, nba: (be[clamp(b, nba)], 0, 0)),
            pl.BlockSpec((1, H, C), lambda b, be, nba: (be[clamp(b, nba)], 0, 0)),
        ],
        out_specs=pl.BlockSpec((R, C // 2), lambda b, be, nba: (clamp(b, nba), 0)),
    )
    return pl.pallas_call(
        _expert_body,
        grid_spec=grid_spec,
        out_shape=jax.ShapeDtypeStruct((NPAD, C // 2), jnp.int32),
    )(block_expert, nb_active, xs, gate_weight, up_weight, down_weight)


_SH_BT = 1024


def _shared_body(x_ref, gw_ref, uw_ref, dw_ref, o_ref):
    xb = x_ref[...].astype(jnp.bfloat16)
    gw = gw_ref[...].astype(jnp.bfloat16)
    uw = uw_ref[...].astype(jnp.bfloat16)
    dw = dw_ref[...].astype(jnp.bfloat16)
    g = lax.dot_general(xb, gw, (((1,), (1,)), ((), ())),
                        preferred_element_type=jnp.float32)
    u = lax.dot_general(xb, uw, (((1,), (1,)), ((), ())),
                        preferred_element_type=jnp.float32)
    h = (g * jax.nn.sigmoid(g) * u).astype(jnp.bfloat16)
    o_ref[...] = lax.dot_general(h, dw, (((1,), (1,)), ((), ())),
                                 preferred_element_type=jnp.float32)


def _run_shared(x_flat, shared_gate_w, shared_up_w, shared_down_w):
    return pl.pallas_call(
        _shared_body,
        grid=(S // _SH_BT,),
        in_specs=[
            pl.BlockSpec((_SH_BT, C), lambda t: (t, 0)),
            pl.BlockSpec((SH, C), lambda t: (0, 0)),
            pl.BlockSpec((SH, C), lambda t: (0, 0)),
            pl.BlockSpec((C, SH), lambda t: (0, 0)),
        ],
        out_specs=pl.BlockSpec((_SH_BT, C), lambda t: (t, 0)),
        out_shape=jax.ShapeDtypeStruct((S, C), jnp.float32),
    )(x_flat, shared_gate_w, shared_up_w, shared_down_w)


_C_TPW = S // NW
_C_TCH = 8
_C_NCH = _C_TPW // _C_TCH


def _combine_body(ys_hbm, pos_hbm, fwb_hbm, sh_hbm, out_hbm,
                  i0, i1, r0, r1, fwb_v, sh_v, o0, o1,
                  g0, g1, s0, s1):
    wid = lax.axis_index("s") * NC + lax.axis_index("c")
    idxs, rows, outs = (i0, i1), (r0, r1), (o0, o1)
    gsem, osem = (g0, g1), (s0, s1)

    def start_gather(c):
        tok0 = wid * _C_TPW + c * _C_TCH
        pltpu.sync_copy(pos_hbm.at[pl.ds(tok0 * K, _C_TCH * K)], idxs[c % 2])
        return pltpu.async_copy(ys_hbm.at[idxs[c % 2]], rows[c % 2],
                                gsem[c % 2])

    gd = [None] * _C_NCH
    od = [None] * _C_NCH
    gd[0] = start_gather(0)
    for c in range(_C_NCH):
        tok0 = wid * _C_TPW + c * _C_TCH
        if c + 1 < _C_NCH:
            gd[c + 1] = start_gather(c + 1)
        pltpu.sync_copy(fwb_hbm.at[pl.ds(tok0, _C_TCH)], fwb_v)
        pltpu.sync_copy(sh_hbm.at[pl.ds(tok0, _C_TCH)], sh_v)
        gd[c].wait()
        if c >= 2:
            od[c - 2].wait()
        rv = rows[c % 2]
        ov = outs[c % 2]
        for t in range(_C_TCH):
            fwj = [fwb_v[t, pl.ds(j * 16, 16)] for j in range(K)]

            def lanes(v, carry2, t=t, fwj=fwj, rv=rv, ov=ov):
                sl = pl.ds(v * L, L)
                slh = pl.ds(C // 2 + v * L, L)
                aa = [sh_v[t, sl], jnp.zeros((L,), jnp.float32)]
                bb_ = [sh_v[t, slh], jnp.zeros((L,), jnp.float32)]
                for j in range(K):
                    pk = plsc.bitcast(rv[t * K + j, sl], jnp.bfloat16)
                    a, bb = plsc.unpack(pk, format=plsc.PackFormat.INTERLEAVED)
                    aa[j % 2] = aa[j % 2] + a * fwj[j]
                    bb_[j % 2] = bb_[j % 2] + bb * fwj[j]
                ov[t, sl] = aa[0] + aa[1]
                ov[t, slh] = bb_[0] + bb_[1]
                return carry2

            lax.fori_loop(0, C // (2 * L), lanes, 0)
        od[c] = pltpu.async_copy(ov, out_hbm.at[pl.ds(tok0, _C_TCH)],
                                 osem[c % 2])
    od[_C_NCH - 2].wait()
    od[_C_NCH - 1].wait()


def _run_combine(ys, pos_flat, fw_bc, shared_out):
    mesh = plsc.VectorSubcoreMesh(core_axis_name="c", subcore_axis_name="s", num_cores=NC, num_subcores=NS)
    return pl.kernel(
        _combine_body,
        compiler_params=pltpu.CompilerParams(needs_layout_passes=False),
        out_type=jax.ShapeDtypeStruct((S, C), jnp.float32),
        mesh=mesh,
        scratch_types=[
            pltpu.VMEM((_C_TCH * K,), jnp.int32),
            pltpu.VMEM((_C_TCH * K,), jnp.int32),
            pltpu.VMEM((_C_TCH * K, C // 2), jnp.int32),
            pltpu.VMEM((_C_TCH * K, C // 2), jnp.int32),
            pltpu.VMEM((_C_TCH, K * 16), jnp.float32),
            pltpu.VMEM((_C_TCH, C), jnp.float32),
            pltpu.VMEM((_C_TCH, C), jnp.float32),
            pltpu.VMEM((_C_TCH, C), jnp.float32),
            pltpu.SemaphoreType.DMA,
            pltpu.SemaphoreType.DMA,
            pltpu.SemaphoreType.DMA,
            pltpu.SemaphoreType.DMA,
        ],
    )(ys, pos_flat, fw_bc, shared_out)


@jax.jit
def kernel(x, router_w, e_bias, gate_weight, up_weight, down_weight,
           shared_gate_w, shared_up_w, shared_down_w):
    x_flat = x.reshape(S, C)
    pos, fw_bc, x_bf16, block_expert, nb_active = _run_router(
        x_flat, router_w, e_bias)
    pos_flat = pos.reshape(SK)
    pos3 = pos.T.reshape(K, NW, _D_TPW).transpose(1, 0, 2)
    xs = _run_dispatch(x_bf16, pos3)
    shared_out = _run_shared(x_flat, shared_gate_w, shared_up_w, shared_down_w)
    ys = _run_experts(xs, gate_weight, up_weight, down_weight,
                      block_expert.reshape(NB), nb_active.reshape(1))
    out = _run_combine(ys, pos_flat, fw_bc, shared_out)
    return out.reshape(B, T, C)
